# baseline (speedup 1.0000x reference)
import functools

import jax
import jax.numpy as jnp
from jax import lax
from jax.experimental import pallas as pl
from jax.experimental.pallas import tpu as pltpu
from jax.experimental.pallas import tpu_sc as plsc

NUM_NODES = 100000
MEM_DIM = 128
TIME_DIM = 16
MSG_DIM = 16
EMBED_DIM = 128
HEADS = 2
DH = EMBED_DIM // HEADS
OUT_CH = 100
N_BATCH = 40000
N_EDGES = 400000
B = 4096

NC = 2
NS = 16
NW = NC * NS
NPAD = 40960
EPAD = 409600
NODE_W = NPAD // NW
EDGE_W = EPAD // NW
CAP_T = 3072
CAP = CAP_T * NW
NSLOT = 8448
TRASH = 8192
PAYW = 144

_mesh = plsc.VectorSubcoreMesh(core_axis_name="c", subcore_axis_name="s")


@functools.partial(
    pl.kernel,
    out_type=[
        jax.ShapeDtypeStruct((NPAD, MEM_DIM), jnp.float32),
        jax.ShapeDtypeStruct((NPAD,), jnp.int32),
        jax.ShapeDtypeStruct((NPAD,), jnp.int32),
    ],
    mesh=_mesh,
    compiler_params=pltpu.CompilerParams(needs_layout_passes=False, use_tc_tiling_on_sc=False),
    scratch_types=[
        pltpu.VMEM((NODE_W,), jnp.int32),
        pltpu.VMEM((128, MEM_DIM), jnp.float32),
        pltpu.VMEM((128,), jnp.int32),
        pltpu.VMEM((2560,), jnp.int32),
        pltpu.VMEM((4, 128), jnp.int32),
        pltpu.VMEM((4, 128), jnp.int32),
        pltpu.VMEM_SHARED((NPAD,), jnp.int32),
        pltpu.SemaphoreType.DMA,
    ],
)
def _k1(mem_hbm, lu_hbm, nid_hbm, srcdst_hbm, z_out, lu_out, sm_out,
        idbuf, zbuf, lubuf, mbuf, nodebuf, jvals, sm_sh, sem):
    c = lax.axis_index("c")
    s = lax.axis_index("s")
    w = s * NC + c
    base = w * NODE_W
    pltpu.sync_copy(nid_hbm.at[pl.ds(base, NODE_W)], idbuf)

    def chunk(i, carry):
        idx = idbuf.at[pl.ds(i * 128, 128)]
        pltpu.async_copy(mem_hbm.at[idx], zbuf, sem).wait()
        pltpu.sync_copy(zbuf, z_out.at[pl.ds(base + i * 128, 128), :])
        pltpu.async_copy(lu_hbm.at[idx], lubuf, sem).wait()
        pltpu.sync_copy(lubuf, lu_out.at[pl.ds(base + i * 128, 128)])
        return carry

    lax.fori_loop(0, NODE_W // 128, chunk, 0)

    @pl.when(c == 0)
    def _():
        def pre(v, carry):
            mbuf[pl.ds(v * 16, 16)] = jnp.full((16,), -1, jnp.int32)
            return carry

        lax.fori_loop(0, 2560 // 16, pre, 0)
        pltpu.sync_copy(mbuf, sm_sh.at[pl.ds(s * 2560, 2560)])
        plsc.subcore_barrier()
        jb = s * 512
        for r in range(4):
            pltpu.sync_copy(srcdst_hbm.at[pl.ds(jb + r * 128, 128)], nodebuf.at[r])
            for v in range(8):
                jvals[r, pl.ds(v * 16, 16)] = lax.iota(jnp.int32, 16) + (jb + r * 128 + v * 16)
            pltpu.sync_copy(jvals.at[r], sm_sh.at[nodebuf.at[r]])
        plsc.subcore_barrier()
        pltpu.sync_copy(sm_sh.at[pl.ds(s * 2560, 2560)], mbuf)
        pltpu.sync_copy(mbuf, sm_out.at[pl.ds(s * 2560, 2560)])


def _k2_body(z_ref, w4_ref, b4_ref, q_ref, kv_ref, sk_ref):
    acc = jnp.dot(z_ref[...], w4_ref[...], preferred_element_type=jnp.float32) + b4_ref[...]
    q_ref[...] = acc[:, 0:128]
    kv_ref[...] = acc[:, 128:384]
    sk_ref[...] = acc[:, 384:512]


_k2 = pl.pallas_call(
    _k2_body,
    grid=(NPAD // 1024,),
    in_specs=[
        pl.BlockSpec((1024, 128), lambda i: (i, 0)),
        pl.BlockSpec((128, 512), lambda i: (0, 0)),
        pl.BlockSpec((1, 512), lambda i: (0, 0)),
    ],
    out_specs=[
        pl.BlockSpec((1024, 128), lambda i: (i, 0)),
        pl.BlockSpec((1024, 256), lambda i: (i, 0)),
        pl.BlockSpec((1024, 128), lambda i: (i, 0)),
    ],
    out_shape=[
        jax.ShapeDtypeStruct((NPAD, 128), jnp.float32),
        jax.ShapeDtypeStruct((NPAD, 256), jnp.float32),
        jax.ShapeDtypeStruct((NPAD, 128), jnp.float32),
    ],
)


@functools.partial(
    pl.kernel,
    out_type=[jax.ShapeDtypeStruct((CAP,), jnp.int32)] * 5,
    mesh=_mesh,
    compiler_params=pltpu.CompilerParams(needs_layout_passes=False, use_tc_tiling_on_sc=False),
    scratch_types=[
        pltpu.VMEM((NPAD,), jnp.int32),
        pltpu.VMEM((640,), jnp.int32),
        pltpu.VMEM((640,), jnp.int32),
        pltpu.VMEM((640,), jnp.int32),
        pltpu.VMEM((EDGE_W,), jnp.int32),
        pltpu.VMEM((EDGE_W,), jnp.int32),
        pltpu.VMEM((EDGE_W,), jnp.int32),
        pltpu.VMEM((EDGE_W,), jnp.int32),
        pltpu.VMEM((EDGE_W,), jnp.int32),
        pltpu.SemaphoreType.DMA,
    ],
)
def _k3a(esrc_hbm, edst_hbm, t_hbm, sm_hbm, osrc, odst, oslot, ot, oeid,
         smb, srcb, dstb, tb, bsrc, bdst, bslot, bt_, beid, sem):
    c = lax.axis_index("c")
    s = lax.axis_index("s")
    w = s * NC + c
    base = w * EDGE_W
    pltpu.sync_copy(sm_hbm, smb)
    iota = lax.iota(jnp.int32, 16)

    def pre(v, carry):
        sl = pl.ds(v * 16, 16)
        z16 = jnp.zeros((16,), jnp.int32)
        bsrc[sl] = z16
        bdst[sl] = z16
        bt_[sl] = z16
        beid[sl] = z16
        bslot[sl] = iota + (TRASH + (v % 8) * 16)
        return carry

    lax.fori_loop(0, CAP_T // 16, pre, 0)

    def batch(i, cnt):
        pltpu.sync_copy(esrc_hbm.at[pl.ds(base + i * 640, 640)], srcb)
        pltpu.sync_copy(edst_hbm.at[pl.ds(base + i * 640, 640)], dstb)
        pltpu.sync_copy(t_hbm.at[pl.ds(base + i * 640, 640)], tb)
        for v in range(40):
            sl = pl.ds(v * 16, 16)
            d = dstb[sl]
            slot = plsc.load_gather(smb, [d])
            m = slot >= 0
            plsc.store_compressed(bslot.at[pl.ds(cnt, 16)], slot, mask=m)
            plsc.store_compressed(bsrc.at[pl.ds(cnt, 16)], srcb[sl], mask=m)
            plsc.store_compressed(bdst.at[pl.ds(cnt, 16)], d, mask=m)
            plsc.store_compressed(bt_.at[pl.ds(cnt, 16)], tb[sl], mask=m)
            plsc.store_compressed(beid.at[pl.ds(cnt, 16)],
                                  iota + (base + i * 640 + v * 16), mask=m)
            cnt = cnt + plsc.all_reduce_population_count(m)[0]
        return cnt

    lax.fori_loop(0, EDGE_W // 640, batch, jnp.int32(0))
    ob = w * CAP_T
    pltpu.sync_copy(bsrc.at[pl.ds(0, CAP_T)], osrc.at[pl.ds(ob, CAP_T)])
    pltpu.sync_copy(bdst.at[pl.ds(0, CAP_T)], odst.at[pl.ds(ob, CAP_T)])
    pltpu.sync_copy(bslot.at[pl.ds(0, CAP_T)], oslot.at[pl.ds(ob, CAP_T)])
    pltpu.sync_copy(bt_.at[pl.ds(0, CAP_T)], ot.at[pl.ds(ob, CAP_T)])
    pltpu.sync_copy(beid.at[pl.ds(0, CAP_T)], oeid.at[pl.ds(ob, CAP_T)])


@functools.partial(
    pl.kernel,
    out_type=jax.ShapeDtypeStruct((CAP,), jnp.float32),
    mesh=_mesh,
    compiler_params=pltpu.CompilerParams(needs_layout_passes=False, use_tc_tiling_on_sc=False),
    scratch_types=[
        pltpu.VMEM((NPAD,), jnp.int32),
        pltpu.VMEM((CAP_T,), jnp.int32),
        pltpu.VMEM((CAP_T,), jnp.int32),
        pltpu.VMEM((CAP_T,), jnp.float32),
        pltpu.SemaphoreType.DMA,
    ],
    name="k3b1_relt",
)
def _k3b1(ssrc, st_, lu_hbm, orelt, lub, src1d, tbuf, reltb, sem):
    c = lax.axis_index("c")
    s = lax.axis_index("s")
    w = s * NC + c
    ob = w * CAP_T
    pltpu.sync_copy(lu_hbm, lub)
    pltpu.sync_copy(ssrc.at[pl.ds(ob, CAP_T)], src1d)
    pltpu.sync_copy(st_.at[pl.ds(ob, CAP_T)], tbuf)

    def rv(v, carry):
        sl = pl.ds(v * 16, 16)
        s16 = src1d[sl]
        lu16 = plsc.load_gather(lub, [s16])
        reltb[sl] = (lu16 - tbuf[sl]).astype(jnp.float32)
        return carry

    lax.fori_loop(0, CAP_T // 16, rv, 0)
    pltpu.sync_copy(reltb, orelt.at[pl.ds(ob, CAP_T)])


_GCH = 64
_NCH = CAP_T // _GCH
_DEP = 3


@functools.partial(
    pl.kernel,
    out_type=[
        jax.ShapeDtypeStruct((CAP, MSG_DIM), jnp.float32),
        jax.ShapeDtypeStruct((CAP, 64), jnp.int32),
        jax.ShapeDtypeStruct((CAP, 128), jnp.int32),
    ],
    mesh=_mesh,
    compiler_params=pltpu.CompilerParams(needs_layout_passes=False, use_tc_tiling_on_sc=False),
    scratch_types=[
        pltpu.VMEM((_NCH, _GCH), jnp.int32),
        pltpu.VMEM((_NCH, _GCH), jnp.int32),
        pltpu.VMEM((_NCH, _GCH), jnp.int32),
    ] + [pltpu.VMEM((_GCH, MSG_DIM), jnp.float32)] * _DEP
      + [pltpu.VMEM((_GCH, 64), jnp.int32)] * _DEP
      + [pltpu.VMEM((_GCH, 128), jnp.int32)] * _DEP
      + [pltpu.SemaphoreType.DMA, pltpu.SemaphoreType.DMA],
    name="k3b2_gather",
)
def _k3b2(ssrc2, sdst2, seid2, q_hbm, kv_hbm, msg_hbm,
          omsg, oq, okv,
          srcb, dstb, eidb,
          m0, m1, m2, q0, q1, q2, k0, k1, k2, gsem, osem):
    c = lax.axis_index("c")
    s = lax.axis_index("s")
    w = s * NC + c
    ob = w * CAP_T
    rb = w * _NCH
    pltpu.sync_copy(ssrc2.at[pl.ds(rb, _NCH), :], srcb)
    pltpu.sync_copy(sdst2.at[pl.ds(rb, _NCH), :], dstb)
    pltpu.sync_copy(seid2.at[pl.ds(rb, _NCH), :], eidb)
    msgt = (m0, m1, m2)
    qt = (q0, q1, q2)
    kvt = (k0, k1, k2)

    def issue_g(r):
        bi = r % _DEP
        return (
            pltpu.async_copy(msg_hbm.at[eidb.at[r]], msgt[bi], gsem),
            pltpu.async_copy(q_hbm.at[dstb.at[r]], qt[bi], gsem),
            pltpu.async_copy(kv_hbm.at[srcb.at[r]], kvt[bi], gsem),
        )

    def issue_o(r):
        bi = r % _DEP
        sl = pl.ds(ob + r * _GCH, _GCH)
        return (
            pltpu.async_copy(msgt[bi], omsg.at[sl, :], osem),
            pltpu.async_copy(qt[bi], oq.at[sl, :], osem),
            pltpu.async_copy(kvt[bi], okv.at[sl, :], osem),
        )

    g = {}
    o = {}
    for r in range(_NCH + 2):
        if r >= _DEP and (r - _DEP) in o:
            for d in o.pop(r - _DEP):
                d.wait()
        if r < _NCH:
            g[r] = issue_g(r)
        if r >= 2:
            for d in g.pop(r - 2):
                d.wait()
            o[r - 2] = issue_o(r - 2)
    for r in sorted(o):
        for d in o[r]:
            d.wait()


def _k4_body(relt_ref, msg_ref, kvs_ref, qs_ref, wt_ref, btb_ref, wet_ref, wem_ref, p_ref):
    relt = relt_ref[...]
    enc = jnp.cos(relt * wt_ref[...] + btb_ref[...])
    ev = jnp.dot(enc, wet_ref[...], preferred_element_type=jnp.float32)
    ev = ev + jnp.dot(msg_ref[...], wem_ref[...], preferred_element_type=jnp.float32)
    kvs = kvs_ref[...].astype(jnp.float32)
    ke = kvs[:, 0:128] + ev
    ve = kvs[:, 128:256] + ev
    prod = qs_ref[...].astype(jnp.float32) * ke
    a0 = jnp.sum(prod[:, 0:64], axis=1, keepdims=True) * 0.125
    a1 = jnp.sum(prod[:, 64:128], axis=1, keepdims=True) * 0.125
    e0 = jnp.exp(a0)
    e1 = jnp.exp(a1)
    vex = ve * jnp.concatenate(
        [jnp.broadcast_to(e0, (1024, 64)), jnp.broadcast_to(e1, (1024, 64))], axis=1)
    lane = lax.broadcasted_iota(jnp.int32, (1024, 16), 1)
    extra = jnp.where(lane == 0, e0, jnp.where(lane == 1, e1, jnp.float32(0)))
    p_ref[...] = jnp.concatenate([vex, extra], axis=1)


_k4 = pl.pallas_call(
    _k4_body,
    grid=(CAP // 1024,),
    in_specs=[
        pl.BlockSpec((1024, 1), lambda i: (i, 0)),
        pl.BlockSpec((1024, MSG_DIM), lambda i: (i, 0)),
        pl.BlockSpec((1024, 256), lambda i: (i, 0)),
        pl.BlockSpec((1024, 128), lambda i: (i, 0)),
        pl.BlockSpec((1, TIME_DIM), lambda i: (0, 0)),
        pl.BlockSpec((1, TIME_DIM), lambda i: (0, 0)),
        pl.BlockSpec((TIME_DIM, 128), lambda i: (0, 0)),
        pl.BlockSpec((MSG_DIM, 128), lambda i: (0, 0)),
    ],
    out_specs=pl.BlockSpec((1024, PAYW), lambda i: (i, 0)),
    out_shape=jax.ShapeDtypeStruct((CAP, PAYW), jnp.float32),
)


@functools.partial(
    pl.kernel,
    out_type=[
        jax.ShapeDtypeStruct((NSLOT, PAYW), jnp.float32),
        jax.ShapeDtypeStruct((NSLOT, PAYW), jnp.float32),
    ],
    mesh=_mesh,
    compiler_params=pltpu.CompilerParams(needs_layout_passes=False, use_tc_tiling_on_sc=False),
    scratch_types=[
        pltpu.VMEM((24, 128), jnp.int32),
        pltpu.VMEM((128, PAYW), jnp.float32),
        pltpu.VMEM((132, PAYW), jnp.float32),
        pltpu.VMEM_SHARED((NSLOT, PAYW), jnp.float32),
        pltpu.SemaphoreType.DMA,
    ],
)
def _k5(p_hbm, slot_hbm, tab0_out, tab1_out, slotb, pbuf, stage, tab_sh, sem):
    c = lax.axis_index("c")
    s = lax.axis_index("s")
    w = s * NC + c
    ob = w * CAP_T
    for r in range(132):
        for v in range(PAYW // 16):
            stage[r, pl.ds(v * 16, 16)] = jnp.zeros((16,), jnp.float32)

    def zs(jj, carry):
        pltpu.sync_copy(stage, tab_sh.at[pl.ds(s * 528 + jj * 132, 132), :])
        return carry

    lax.fori_loop(0, 4, zs, 0)
    plsc.subcore_barrier()

    def r24(r, carry):
        pltpu.sync_copy(slot_hbm.at[pl.ds(ob + r * 128, 128)], slotb.at[r])
        pltpu.sync_copy(p_hbm.at[pl.ds(ob + r * 128, 128), :], pbuf)
        pltpu.sync_copy(pbuf, tab_sh.at[slotb.at[r]], add=True)
        return carry

    lax.fori_loop(0, CAP_T // 128, r24, 0)
    plsc.subcore_barrier()

    def dmp(jj, carry):
        pltpu.sync_copy(tab_sh.at[pl.ds(s * 528 + jj * 132, 132), :], stage)

        @pl.when(c == 0)
        def _():
            pltpu.sync_copy(stage, tab0_out.at[pl.ds(s * 528 + jj * 132, 132), :])

        @pl.when(c == 1)
        def _():
            pltpu.sync_copy(stage, tab1_out.at[pl.ds(s * 528 + jj * 132, 132), :])

        return carry

    lax.fori_loop(0, 4, dmp, 0)


@functools.partial(
    pl.kernel,
    out_type=[
        jax.ShapeDtypeStruct((2 * B, PAYW), jnp.float32),
        jax.ShapeDtypeStruct((2 * B, PAYW), jnp.float32),
        jax.ShapeDtypeStruct((2 * B, 128), jnp.float32),
    ],
    mesh=_mesh,
    compiler_params=pltpu.CompilerParams(needs_layout_passes=False, use_tc_tiling_on_sc=False),
    scratch_types=[
        pltpu.VMEM((NPAD,), jnp.int32),
        pltpu.VMEM((2, 128), jnp.int32),
        pltpu.VMEM((2, 128), jnp.int32),
        pltpu.VMEM((128, PAYW), jnp.float32),
        pltpu.VMEM((128, 128), jnp.float32),
        pltpu.SemaphoreType.DMA,
    ],
)
def _k6(sm_hbm, srcdst_hbm, tab0_hbm, tab1_hbm, skip_hbm, g0, g1, sk,
        smb, nb, sb, gt, skt, sem):
    c = lax.axis_index("c")
    s = lax.axis_index("s")
    w = s * NC + c
    ob = w * (2 * B // NW)
    pltpu.sync_copy(sm_hbm, smb)
    for r in range(2):
        pltpu.sync_copy(srcdst_hbm.at[pl.ds(ob + r * 128, 128)], nb.at[r])
        for v in range(8):
            n16 = nb[r, pl.ds(v * 16, 16)]
            sb[r, pl.ds(v * 16, 16)] = plsc.load_gather(smb, [n16])
        pltpu.async_copy(tab0_hbm.at[sb.at[r]], gt, sem).wait()
        pltpu.sync_copy(gt, g0.at[pl.ds(ob + r * 128, 128), :])
        pltpu.async_copy(tab1_hbm.at[sb.at[r]], gt, sem).wait()
        pltpu.sync_copy(gt, g1.at[pl.ds(ob + r * 128, 128), :])
        pltpu.async_copy(skip_hbm.at[nb.at[r]], skt, sem).wait()
        pltpu.sync_copy(skt, sk.at[pl.ds(ob + r * 128, 128), :])


_BLK7 = 512


def _k7_body(g0s, g1s, sks, g0d, g1d, skd, wsrc_ref, wdst_ref, bh_ref, wout_ref,
             bout_ref, y_ref):
    def node_out(a, b, sk):
        num = a[:, 0:128] + b[:, 0:128]
        d0 = a[:, 128:129] + b[:, 128:129]
        d1 = a[:, 129:130] + b[:, 129:130]
        den = jnp.concatenate(
            [jnp.broadcast_to(d0, (_BLK7, 64)), jnp.broadcast_to(d1, (_BLK7, 64))],
            axis=1)
        return num / (den + 1e-16) + sk

    os_ = node_out(g0s[...], g1s[...], sks[...])
    od_ = node_out(g0d[...], g1d[...], skd[...])
    h = os_ @ wsrc_ref[...] + od_ @ wdst_ref[...] + bh_ref[...]
    h = jnp.maximum(h, 0.0)
    y_ref[...] = h @ wout_ref[...] + bout_ref[...]


_k7 = pl.pallas_call(
    _k7_body,
    grid=(B // _BLK7,),
    in_specs=[
        pl.BlockSpec((_BLK7, PAYW), lambda i: (i, 0)),
        pl.BlockSpec((_BLK7, PAYW), lambda i: (i, 0)),
        pl.BlockSpec((_BLK7, 128), lambda i: (i, 0)),
        pl.BlockSpec((_BLK7, PAYW), lambda i: (i, 0)),
        pl.BlockSpec((_BLK7, PAYW), lambda i: (i, 0)),
        pl.BlockSpec((_BLK7, 128), lambda i: (i, 0)),
        pl.BlockSpec((128, 128), lambda i: (0, 0)),
        pl.BlockSpec((128, 128), lambda i: (0, 0)),
        pl.BlockSpec((1, 128), lambda i: (0, 0)),
        pl.BlockSpec((128, OUT_CH), lambda i: (0, 0)),
        pl.BlockSpec((1, OUT_CH), lambda i: (0, 0)),
    ],
    out_specs=pl.BlockSpec((_BLK7, OUT_CH), lambda i: (i, 0)),
    out_shape=jax.ShapeDtypeStruct((B, OUT_CH), jnp.float32),
)


def kernel(n_id, edge_index, t, msg, src, dst, memory, last_update, Wt, bt, Wq, bq, Wk, bk, Wv, bv, We, Wskip, bskip, Wsrc, Wdst, bh, Wout, bout):
    nid_p = jnp.concatenate([n_id, jnp.zeros((NPAD - N_BATCH,), jnp.int32)])
    srcdst = jnp.concatenate([src, dst])
    z, lu, slotmap = _k1(memory, last_update, nid_p, srcdst)

    w4 = jnp.concatenate([Wq, Wk, Wv, Wskip], axis=1)
    b4 = jnp.concatenate([bq, bk, bv, bskip])[None, :]
    q, kv, skip = _k2(z, w4, b4)

    epad = jnp.full((EPAD - N_EDGES,), N_BATCH, jnp.int32)
    esrc = jnp.concatenate([edge_index[0], epad])
    edst = jnp.concatenate([edge_index[1], epad])
    tp = jnp.concatenate([t, jnp.zeros((EPAD - N_EDGES,), jnp.int32)])
    ssrc, sdst, sslot, st_, seid = _k3a(esrc, edst, tp, slotmap)

    relt = _k3b1(ssrc, st_, lu)
    ssrc2 = ssrc.reshape(CAP // _GCH, _GCH)
    sdst2 = sdst.reshape(CAP // _GCH, _GCH)
    seid2 = seid.reshape(CAP // _GCH, _GCH)
    qp = lax.bitcast_convert_type(
        q.astype(jnp.bfloat16).reshape(NPAD, 64, 2), jnp.int32)
    kvp = lax.bitcast_convert_type(
        kv.astype(jnp.bfloat16).reshape(NPAD, 128, 2), jnp.int32)
    msgs, qsp, kvsp = _k3b2(ssrc2, sdst2, seid2, qp, kvp, msg)
    qs = lax.bitcast_convert_type(qsp, jnp.bfloat16).reshape(CAP, 128)
    kvs = lax.bitcast_convert_type(kvsp, jnp.bfloat16).reshape(CAP, 256)

    p = _k4(relt[:, None], msgs, kvs, qs, Wt, bt[None, :], We[:TIME_DIM], We[TIME_DIM:])

    tab0, tab1 = _k5(p, sslot)

    g0, g1, sk = _k6(slotmap, srcdst, tab0, tab1, skip)

    y = _k7(g0[:B], g1[:B], sk[:B], g0[B:], g1[B:], sk[B:],
            Wsrc, Wdst, bh[None, :], Wout, bout[None, :])
    return y


# speedup vs baseline: 1.3736x; 1.3736x over previous
import functools

import jax
import jax.numpy as jnp
from jax import lax
from jax.experimental import pallas as pl
from jax.experimental.pallas import tpu as pltpu
from jax.experimental.pallas import tpu_sc as plsc

NUM_NODES = 100000
MEM_DIM = 128
TIME_DIM = 16
MSG_DIM = 16
EMBED_DIM = 128
HEADS = 2
DH = EMBED_DIM // HEADS
OUT_CH = 100
N_BATCH = 40000
N_EDGES = 400000
B = 4096

NC = 2
NS = 16
NW = NC * NS
NPAD = 40960
EPAD = 409600
NODE_W = NPAD // NW
EDGE_W = EPAD // NW
CAP_T = 3072
CAP = CAP_T * NW
NSLOT = 8448
TRASH = 8192
PAYW = 144

_mesh = plsc.VectorSubcoreMesh(core_axis_name="c", subcore_axis_name="s")


@functools.partial(
    pl.kernel,
    out_type=[
        jax.ShapeDtypeStruct((NPAD, MEM_DIM), jnp.float32),
        jax.ShapeDtypeStruct((NPAD,), jnp.int32),
        jax.ShapeDtypeStruct((NPAD,), jnp.int32),
    ],
    mesh=_mesh,
    compiler_params=pltpu.CompilerParams(needs_layout_passes=False, use_tc_tiling_on_sc=False),
    scratch_types=[
        pltpu.VMEM((NODE_W,), jnp.int32),
        pltpu.VMEM((128, MEM_DIM), jnp.float32),
        pltpu.VMEM((128,), jnp.int32),
        pltpu.VMEM((2560,), jnp.int32),
        pltpu.VMEM((4, 128), jnp.int32),
        pltpu.VMEM((4, 128), jnp.int32),
        pltpu.VMEM_SHARED((NPAD,), jnp.int32),
        pltpu.SemaphoreType.DMA,
    ],
)
def _k1(mem_hbm, lu_hbm, nid_hbm, srcdst_hbm, z_out, lu_out, sm_out,
        idbuf, zbuf, lubuf, mbuf, nodebuf, jvals, sm_sh, sem):
    c = lax.axis_index("c")
    s = lax.axis_index("s")
    w = s * NC + c
    base = w * NODE_W
    pltpu.sync_copy(nid_hbm.at[pl.ds(base, NODE_W)], idbuf)

    def chunk(i, carry):
        idx = idbuf.at[pl.ds(i * 128, 128)]
        pltpu.async_copy(mem_hbm.at[idx], zbuf, sem).wait()
        pltpu.sync_copy(zbuf, z_out.at[pl.ds(base + i * 128, 128), :])
        pltpu.async_copy(lu_hbm.at[idx], lubuf, sem).wait()
        pltpu.sync_copy(lubuf, lu_out.at[pl.ds(base + i * 128, 128)])
        return carry

    lax.fori_loop(0, NODE_W // 128, chunk, 0)

    @pl.when(c == 0)
    def _():
        def pre(v, carry):
            mbuf[pl.ds(v * 16, 16)] = jnp.full((16,), -1, jnp.int32)
            return carry

        lax.fori_loop(0, 2560 // 16, pre, 0)
        pltpu.sync_copy(mbuf, sm_sh.at[pl.ds(s * 2560, 2560)])
        plsc.subcore_barrier()
        jb = s * 512
        for r in range(4):
            pltpu.sync_copy(srcdst_hbm.at[pl.ds(jb + r * 128, 128)], nodebuf.at[r])
            for v in range(8):
                jvals[r, pl.ds(v * 16, 16)] = lax.iota(jnp.int32, 16) + (jb + r * 128 + v * 16)
            pltpu.sync_copy(jvals.at[r], sm_sh.at[nodebuf.at[r]])
        plsc.subcore_barrier()
        pltpu.sync_copy(sm_sh.at[pl.ds(s * 2560, 2560)], mbuf)
        pltpu.sync_copy(mbuf, sm_out.at[pl.ds(s * 2560, 2560)])


def _k2_body(z_ref, w4_ref, b4_ref, q_ref, kv_ref, sk_ref):
    acc = jnp.dot(z_ref[...], w4_ref[...], preferred_element_type=jnp.float32) + b4_ref[...]
    q_ref[...] = acc[:, 0:128]
    kv_ref[...] = acc[:, 128:384]
    sk_ref[...] = acc[:, 384:512]


_k2 = pl.pallas_call(
    _k2_body,
    grid=(NPAD // 1024,),
    in_specs=[
        pl.BlockSpec((1024, 128), lambda i: (i, 0)),
        pl.BlockSpec((128, 512), lambda i: (0, 0)),
        pl.BlockSpec((1, 512), lambda i: (0, 0)),
    ],
    out_specs=[
        pl.BlockSpec((1024, 128), lambda i: (i, 0)),
        pl.BlockSpec((1024, 256), lambda i: (i, 0)),
        pl.BlockSpec((1024, 128), lambda i: (i, 0)),
    ],
    out_shape=[
        jax.ShapeDtypeStruct((NPAD, 128), jnp.float32),
        jax.ShapeDtypeStruct((NPAD, 256), jnp.float32),
        jax.ShapeDtypeStruct((NPAD, 128), jnp.float32),
    ],
)


@functools.partial(
    pl.kernel,
    out_type=[jax.ShapeDtypeStruct((CAP,), jnp.int32)] * 5,
    mesh=_mesh,
    compiler_params=pltpu.CompilerParams(needs_layout_passes=False, use_tc_tiling_on_sc=False),
    scratch_types=[
        pltpu.VMEM((NPAD,), jnp.int32),
        pltpu.VMEM((640,), jnp.int32),
        pltpu.VMEM((640,), jnp.int32),
        pltpu.VMEM((640,), jnp.int32),
        pltpu.VMEM((EDGE_W,), jnp.int32),
        pltpu.VMEM((EDGE_W,), jnp.int32),
        pltpu.VMEM((EDGE_W,), jnp.int32),
        pltpu.VMEM((EDGE_W,), jnp.int32),
        pltpu.VMEM((EDGE_W,), jnp.int32),
        pltpu.SemaphoreType.DMA,
    ],
)
def _k3a(esrc_hbm, edst_hbm, t_hbm, sm_hbm, osrc, odst, oslot, ot, oeid,
         smb, srcb, dstb, tb, bsrc, bdst, bslot, bt_, beid, sem):
    c = lax.axis_index("c")
    s = lax.axis_index("s")
    w = s * NC + c
    base = w * EDGE_W
    pltpu.sync_copy(sm_hbm, smb)
    iota = lax.iota(jnp.int32, 16)

    def pre(v, carry):
        sl = pl.ds(v * 16, 16)
        z16 = jnp.zeros((16,), jnp.int32)
        bsrc[sl] = z16
        bdst[sl] = z16
        bt_[sl] = z16
        beid[sl] = z16
        bslot[sl] = iota + (TRASH + (v % 8) * 16)
        return carry

    lax.fori_loop(0, CAP_T // 16, pre, 0)

    def batch(i, cnt):
        pltpu.sync_copy(esrc_hbm.at[pl.ds(base + i * 640, 640)], srcb)
        pltpu.sync_copy(edst_hbm.at[pl.ds(base + i * 640, 640)], dstb)
        pltpu.sync_copy(t_hbm.at[pl.ds(base + i * 640, 640)], tb)
        for v in range(40):
            sl = pl.ds(v * 16, 16)
            d = dstb[sl]
            slot = plsc.load_gather(smb, [d])
            m = slot >= 0
            plsc.store_compressed(bslot.at[pl.ds(cnt, 16)], slot, mask=m)
            plsc.store_compressed(bsrc.at[pl.ds(cnt, 16)], srcb[sl], mask=m)
            plsc.store_compressed(bdst.at[pl.ds(cnt, 16)], d, mask=m)
            plsc.store_compressed(bt_.at[pl.ds(cnt, 16)], tb[sl], mask=m)
            plsc.store_compressed(beid.at[pl.ds(cnt, 16)],
                                  iota + (base + i * 640 + v * 16), mask=m)
            cnt = cnt + plsc.all_reduce_population_count(m)[0]
        return cnt

    lax.fori_loop(0, EDGE_W // 640, batch, jnp.int32(0))
    ob = w * CAP_T
    pltpu.sync_copy(bsrc.at[pl.ds(0, CAP_T)], osrc.at[pl.ds(ob, CAP_T)])
    pltpu.sync_copy(bdst.at[pl.ds(0, CAP_T)], odst.at[pl.ds(ob, CAP_T)])
    pltpu.sync_copy(bslot.at[pl.ds(0, CAP_T)], oslot.at[pl.ds(ob, CAP_T)])
    pltpu.sync_copy(bt_.at[pl.ds(0, CAP_T)], ot.at[pl.ds(ob, CAP_T)])
    pltpu.sync_copy(beid.at[pl.ds(0, CAP_T)], oeid.at[pl.ds(ob, CAP_T)])


@functools.partial(
    pl.kernel,
    out_type=jax.ShapeDtypeStruct((CAP,), jnp.float32),
    mesh=_mesh,
    compiler_params=pltpu.CompilerParams(needs_layout_passes=False, use_tc_tiling_on_sc=False),
    scratch_types=[
        pltpu.VMEM((NPAD,), jnp.int32),
        pltpu.VMEM((CAP_T,), jnp.int32),
        pltpu.VMEM((CAP_T,), jnp.int32),
        pltpu.VMEM((CAP_T,), jnp.float32),
        pltpu.SemaphoreType.DMA,
    ],
    name="k3b1_relt",
)
def _k3b1(ssrc, st_, lu_hbm, orelt, lub, src1d, tbuf, reltb, sem):
    c = lax.axis_index("c")
    s = lax.axis_index("s")
    w = s * NC + c
    ob = w * CAP_T
    pltpu.sync_copy(lu_hbm, lub)
    pltpu.sync_copy(ssrc.at[pl.ds(ob, CAP_T)], src1d)
    pltpu.sync_copy(st_.at[pl.ds(ob, CAP_T)], tbuf)

    def rv(v, carry):
        sl = pl.ds(v * 16, 16)
        s16 = src1d[sl]
        lu16 = plsc.load_gather(lub, [s16])
        reltb[sl] = (lu16 - tbuf[sl]).astype(jnp.float32)
        return carry

    lax.fori_loop(0, CAP_T // 16, rv, 0)
    pltpu.sync_copy(reltb, orelt.at[pl.ds(ob, CAP_T)])


_GCH = 32
_NCH = CAP_T // _GCH
_DEP = 6


@functools.partial(
    pl.kernel,
    out_type=[
        jax.ShapeDtypeStruct((CAP, MSG_DIM), jnp.float32),
        jax.ShapeDtypeStruct((CAP, 128), jnp.float32),
        jax.ShapeDtypeStruct((CAP, 256), jnp.float32),
    ],
    mesh=_mesh,
    compiler_params=pltpu.CompilerParams(needs_layout_passes=False, use_tc_tiling_on_sc=False),
    scratch_types=[
        pltpu.VMEM((_NCH, _GCH), jnp.int32),
        pltpu.VMEM((_NCH, _GCH), jnp.int32),
        pltpu.VMEM((_NCH, _GCH), jnp.int32),
    ] + [pltpu.VMEM((_GCH, MSG_DIM), jnp.float32)] * _DEP
      + [pltpu.VMEM((_GCH, 128), jnp.float32)] * _DEP
      + [pltpu.VMEM((_GCH, 256), jnp.float32)] * _DEP
      + [pltpu.SemaphoreType.DMA, pltpu.SemaphoreType.DMA],
    name="k3b2_gather",
)
def _k3b2(ssrc2, sdst2, seid2, q_hbm, kv_hbm, msg_hbm,
          omsg, oq, okv,
          srcb, dstb, eidb,
          m0, m1, m2, m3, m4, m5,
          q0, q1, q2, q3, q4, q5,
          k0, k1, k2, k3, k4, k5, gsem, osem):
    c = lax.axis_index("c")
    s = lax.axis_index("s")
    w = s * NC + c
    ob = w * CAP_T
    rb = w * _NCH
    pltpu.sync_copy(ssrc2.at[pl.ds(rb, _NCH), :], srcb)
    pltpu.sync_copy(sdst2.at[pl.ds(rb, _NCH), :], dstb)
    pltpu.sync_copy(seid2.at[pl.ds(rb, _NCH), :], eidb)
    msgt = (m0, m1, m2, m3, m4, m5)
    qt = (q0, q1, q2, q3, q4, q5)
    kvt = (k0, k1, k2, k3, k4, k5)

    def issue_g(r):
        bi = r % _DEP
        return (
            pltpu.async_copy(msg_hbm.at[eidb.at[r]], msgt[bi], gsem),
            pltpu.async_copy(q_hbm.at[dstb.at[r]], qt[bi], gsem),
            pltpu.async_copy(kv_hbm.at[srcb.at[r]], kvt[bi], gsem),
        )

    def issue_o(r):
        bi = r % _DEP
        sl = pl.ds(ob + r * _GCH, _GCH)
        return (
            pltpu.async_copy(msgt[bi], omsg.at[sl, :], osem),
            pltpu.async_copy(qt[bi], oq.at[sl, :], osem),
            pltpu.async_copy(kvt[bi], okv.at[sl, :], osem),
        )

    g = {}
    o = {}
    for r in range(_NCH + 4):
        if (r - _DEP) in o:
            for d in o.pop(r - _DEP):
                d.wait()
        if r < _NCH:
            g[r] = issue_g(r)
        if (r - 4) in g:
            for d in g.pop(r - 4):
                d.wait()
            o[r - 4] = issue_o(r - 4)
    for r in sorted(o):
        for d in o[r]:
            d.wait()


def _k4_body(relt_ref, msg_ref, kvs_ref, qs_ref, wt_ref, btb_ref, wet_ref, wem_ref, p_ref):
    relt = relt_ref[...]
    enc = jnp.cos(relt * wt_ref[...] + btb_ref[...])
    ev = jnp.dot(enc, wet_ref[...], preferred_element_type=jnp.float32)
    ev = ev + jnp.dot(msg_ref[...], wem_ref[...], preferred_element_type=jnp.float32)
    kvs = kvs_ref[...]
    ke = kvs[:, 0:128] + ev
    ve = kvs[:, 128:256] + ev
    prod = qs_ref[...] * ke
    a0 = jnp.sum(prod[:, 0:64], axis=1, keepdims=True) * 0.125
    a1 = jnp.sum(prod[:, 64:128], axis=1, keepdims=True) * 0.125
    e0 = jnp.exp(a0)
    e1 = jnp.exp(a1)
    vex = ve * jnp.concatenate(
        [jnp.broadcast_to(e0, (1024, 64)), jnp.broadcast_to(e1, (1024, 64))], axis=1)
    lane = lax.broadcasted_iota(jnp.int32, (1024, 16), 1)
    extra = jnp.where(lane == 0, e0, jnp.where(lane == 1, e1, jnp.float32(0)))
    p_ref[...] = jnp.concatenate([vex, extra], axis=1)


_k4 = pl.pallas_call(
    _k4_body,
    grid=(CAP // 1024,),
    in_specs=[
        pl.BlockSpec((1024, 1), lambda i: (i, 0)),
        pl.BlockSpec((1024, MSG_DIM), lambda i: (i, 0)),
        pl.BlockSpec((1024, 256), lambda i: (i, 0)),
        pl.BlockSpec((1024, 128), lambda i: (i, 0)),
        pl.BlockSpec((1, TIME_DIM), lambda i: (0, 0)),
        pl.BlockSpec((1, TIME_DIM), lambda i: (0, 0)),
        pl.BlockSpec((TIME_DIM, 128), lambda i: (0, 0)),
        pl.BlockSpec((MSG_DIM, 128), lambda i: (0, 0)),
    ],
    out_specs=pl.BlockSpec((1024, PAYW), lambda i: (i, 0)),
    out_shape=jax.ShapeDtypeStruct((CAP, PAYW), jnp.float32),
)


@functools.partial(
    pl.kernel,
    out_type=[
        jax.ShapeDtypeStruct((NSLOT, PAYW), jnp.float32),
        jax.ShapeDtypeStruct((NSLOT, PAYW), jnp.float32),
    ],
    mesh=_mesh,
    compiler_params=pltpu.CompilerParams(needs_layout_passes=False, use_tc_tiling_on_sc=False),
    scratch_types=[
        pltpu.VMEM((24, 128), jnp.int32),
        pltpu.VMEM((128, PAYW), jnp.float32),
        pltpu.VMEM((132, PAYW), jnp.float32),
        pltpu.VMEM_SHARED((NSLOT, PAYW), jnp.float32),
        pltpu.SemaphoreType.DMA,
    ],
)
def _k5(p_hbm, slot_hbm, tab0_out, tab1_out, slotb, pbuf, stage, tab_sh, sem):
    c = lax.axis_index("c")
    s = lax.axis_index("s")
    w = s * NC + c
    ob = w * CAP_T
    for r in range(132):
        for v in range(PAYW // 16):
            stage[r, pl.ds(v * 16, 16)] = jnp.zeros((16,), jnp.float32)

    def zs(jj, carry):
        pltpu.sync_copy(stage, tab_sh.at[pl.ds(s * 528 + jj * 132, 132), :])
        return carry

    lax.fori_loop(0, 4, zs, 0)
    plsc.subcore_barrier()

    def r24(r, carry):
        pltpu.sync_copy(slot_hbm.at[pl.ds(ob + r * 128, 128)], slotb.at[r])
        pltpu.sync_copy(p_hbm.at[pl.ds(ob + r * 128, 128), :], pbuf)
        pltpu.sync_copy(pbuf, tab_sh.at[slotb.at[r]], add=True)
        return carry

    lax.fori_loop(0, CAP_T // 128, r24, 0)
    plsc.subcore_barrier()

    def dmp(jj, carry):
        pltpu.sync_copy(tab_sh.at[pl.ds(s * 528 + jj * 132, 132), :], stage)

        @pl.when(c == 0)
        def _():
            pltpu.sync_copy(stage, tab0_out.at[pl.ds(s * 528 + jj * 132, 132), :])

        @pl.when(c == 1)
        def _():
            pltpu.sync_copy(stage, tab1_out.at[pl.ds(s * 528 + jj * 132, 132), :])

        return carry

    lax.fori_loop(0, 4, dmp, 0)


@functools.partial(
    pl.kernel,
    out_type=[
        jax.ShapeDtypeStruct((2 * B, PAYW), jnp.float32),
        jax.ShapeDtypeStruct((2 * B, PAYW), jnp.float32),
        jax.ShapeDtypeStruct((2 * B, 128), jnp.float32),
    ],
    mesh=_mesh,
    compiler_params=pltpu.CompilerParams(needs_layout_passes=False, use_tc_tiling_on_sc=False),
    scratch_types=[
        pltpu.VMEM((NPAD,), jnp.int32),
        pltpu.VMEM((2, 128), jnp.int32),
        pltpu.VMEM((2, 128), jnp.int32),
        pltpu.VMEM((128, PAYW), jnp.float32),
        pltpu.VMEM((128, 128), jnp.float32),
        pltpu.SemaphoreType.DMA,
    ],
)
def _k6(sm_hbm, srcdst_hbm, tab0_hbm, tab1_hbm, skip_hbm, g0, g1, sk,
        smb, nb, sb, gt, skt, sem):
    c = lax.axis_index("c")
    s = lax.axis_index("s")
    w = s * NC + c
    ob = w * (2 * B // NW)
    pltpu.sync_copy(sm_hbm, smb)
    for r in range(2):
        pltpu.sync_copy(srcdst_hbm.at[pl.ds(ob + r * 128, 128)], nb.at[r])
        for v in range(8):
            n16 = nb[r, pl.ds(v * 16, 16)]
            sb[r, pl.ds(v * 16, 16)] = plsc.load_gather(smb, [n16])
        pltpu.async_copy(tab0_hbm.at[sb.at[r]], gt, sem).wait()
        pltpu.sync_copy(gt, g0.at[pl.ds(ob + r * 128, 128), :])
        pltpu.async_copy(tab1_hbm.at[sb.at[r]], gt, sem).wait()
        pltpu.sync_copy(gt, g1.at[pl.ds(ob + r * 128, 128), :])
        pltpu.async_copy(skip_hbm.at[nb.at[r]], skt, sem).wait()
        pltpu.sync_copy(skt, sk.at[pl.ds(ob + r * 128, 128), :])


_BLK7 = 512


def _k7_body(g0s, g1s, sks, g0d, g1d, skd, wsrc_ref, wdst_ref, bh_ref, wout_ref,
             bout_ref, y_ref):
    def node_out(a, b, sk):
        num = a[:, 0:128] + b[:, 0:128]
        d0 = a[:, 128:129] + b[:, 128:129]
        d1 = a[:, 129:130] + b[:, 129:130]
        den = jnp.concatenate(
            [jnp.broadcast_to(d0, (_BLK7, 64)), jnp.broadcast_to(d1, (_BLK7, 64))],
            axis=1)
        return num / (den + 1e-16) + sk

    os_ = node_out(g0s[...], g1s[...], sks[...])
    od_ = node_out(g0d[...], g1d[...], skd[...])
    h = os_ @ wsrc_ref[...] + od_ @ wdst_ref[...] + bh_ref[...]
    h = jnp.maximum(h, 0.0)
    y_ref[...] = h @ wout_ref[...] + bout_ref[...]


_k7 = pl.pallas_call(
    _k7_body,
    grid=(B // _BLK7,),
    in_specs=[
        pl.BlockSpec((_BLK7, PAYW), lambda i: (i, 0)),
        pl.BlockSpec((_BLK7, PAYW), lambda i: (i, 0)),
        pl.BlockSpec((_BLK7, 128), lambda i: (i, 0)),
        pl.BlockSpec((_BLK7, PAYW), lambda i: (i, 0)),
        pl.BlockSpec((_BLK7, PAYW), lambda i: (i, 0)),
        pl.BlockSpec((_BLK7, 128), lambda i: (i, 0)),
        pl.BlockSpec((128, 128), lambda i: (0, 0)),
        pl.BlockSpec((128, 128), lambda i: (0, 0)),
        pl.BlockSpec((1, 128), lambda i: (0, 0)),
        pl.BlockSpec((128, OUT_CH), lambda i: (0, 0)),
        pl.BlockSpec((1, OUT_CH), lambda i: (0, 0)),
    ],
    out_specs=pl.BlockSpec((_BLK7, OUT_CH), lambda i: (i, 0)),
    out_shape=jax.ShapeDtypeStruct((B, OUT_CH), jnp.float32),
)


def kernel(n_id, edge_index, t, msg, src, dst, memory, last_update, Wt, bt, Wq, bq, Wk, bk, Wv, bv, We, Wskip, bskip, Wsrc, Wdst, bh, Wout, bout):
    nid_p = jnp.concatenate([n_id, jnp.zeros((NPAD - N_BATCH,), jnp.int32)])
    srcdst = jnp.concatenate([src, dst])
    z, lu, slotmap = _k1(memory, last_update, nid_p, srcdst)

    w4 = jnp.concatenate([Wq, Wk, Wv, Wskip], axis=1)
    b4 = jnp.concatenate([bq, bk, bv, bskip])[None, :]
    q, kv, skip = _k2(z, w4, b4)

    epad = jnp.full((EPAD - N_EDGES,), N_BATCH, jnp.int32)
    esrc = jnp.concatenate([edge_index[0], epad])
    edst = jnp.concatenate([edge_index[1], epad])
    tp = jnp.concatenate([t, jnp.zeros((EPAD - N_EDGES,), jnp.int32)])
    ssrc, sdst, sslot, st_, seid = _k3a(esrc, edst, tp, slotmap)

    relt = _k3b1(ssrc, st_, lu)
    ssrc2 = ssrc.reshape(CAP // _GCH, _GCH)
    sdst2 = sdst.reshape(CAP // _GCH, _GCH)
    seid2 = seid.reshape(CAP // _GCH, _GCH)
    msgs, qs, kvs = _k3b2(ssrc2, sdst2, seid2, q, kv, msg)

    p = _k4(relt[:, None], msgs, kvs, qs, Wt, bt[None, :], We[:TIME_DIM], We[TIME_DIM:])

    tab0, tab1 = _k5(p, sslot)

    g0, g1, sk = _k6(slotmap, srcdst, tab0, tab1, skip)

    y = _k7(g0[:B], g1[:B], sk[:B], g0[B:], g1[B:], sk[B:],
            Wsrc, Wdst, bh[None, :], Wout, bout[None, :])
    return y


# speedup vs baseline: 2.5915x; 1.8867x over previous
import functools

import jax
import jax.numpy as jnp
from jax import lax
from jax.experimental import pallas as pl
from jax.experimental.pallas import tpu as pltpu
from jax.experimental.pallas import tpu_sc as plsc

NUM_NODES = 100000
MEM_DIM = 128
TIME_DIM = 16
MSG_DIM = 16
EMBED_DIM = 128
HEADS = 2
DH = EMBED_DIM // HEADS
OUT_CH = 100
N_BATCH = 40000
N_EDGES = 400000
B = 4096

NC = 2
NS = 16
NW = NC * NS
NPAD = 40960
EPAD = 409600
NODE_W = NPAD // NW
EDGE_W = EPAD // NW
CAP_T = 3072
CAP = CAP_T * NW
NSLOT = 8448
TRASH = 8192
PAYW = 144

_mesh = plsc.VectorSubcoreMesh(core_axis_name="c", subcore_axis_name="s")


@functools.partial(
    pl.kernel,
    out_type=[
        jax.ShapeDtypeStruct((NPAD, MEM_DIM), jnp.float32),
        jax.ShapeDtypeStruct((NPAD,), jnp.int32),
        jax.ShapeDtypeStruct((NPAD,), jnp.int32),
    ],
    mesh=_mesh,
    compiler_params=pltpu.CompilerParams(needs_layout_passes=False, use_tc_tiling_on_sc=False),
    scratch_types=[
        pltpu.VMEM((NODE_W,), jnp.int32),
        pltpu.VMEM((128, MEM_DIM), jnp.float32),
        pltpu.VMEM((128,), jnp.int32),
        pltpu.VMEM((2560,), jnp.int32),
        pltpu.VMEM((4, 128), jnp.int32),
        pltpu.VMEM((4, 128), jnp.int32),
        pltpu.VMEM_SHARED((NPAD,), jnp.int32),
        pltpu.SemaphoreType.DMA,
    ],
)
def _k1(mem_hbm, lu_hbm, nid_hbm, srcdst_hbm, z_out, lu_out, sm_out,
        idbuf, zbuf, lubuf, mbuf, nodebuf, jvals, sm_sh, sem):
    c = lax.axis_index("c")
    s = lax.axis_index("s")
    w = s * NC + c
    base = w * NODE_W
    pltpu.sync_copy(nid_hbm.at[pl.ds(base, NODE_W)], idbuf)

    def chunk(i, carry):
        idx = idbuf.at[pl.ds(i * 128, 128)]
        pltpu.async_copy(mem_hbm.at[idx], zbuf, sem).wait()
        pltpu.sync_copy(zbuf, z_out.at[pl.ds(base + i * 128, 128), :])
        pltpu.async_copy(lu_hbm.at[idx], lubuf, sem).wait()
        pltpu.sync_copy(lubuf, lu_out.at[pl.ds(base + i * 128, 128)])
        return carry

    lax.fori_loop(0, NODE_W // 128, chunk, 0)

    @pl.when(c == 0)
    def _():
        def pre(v, carry):
            mbuf[pl.ds(v * 16, 16)] = jnp.full((16,), -1, jnp.int32)
            return carry

        lax.fori_loop(0, 2560 // 16, pre, 0)
        pltpu.sync_copy(mbuf, sm_sh.at[pl.ds(s * 2560, 2560)])
        plsc.subcore_barrier()
        jb = s * 512
        for r in range(4):
            pltpu.sync_copy(srcdst_hbm.at[pl.ds(jb + r * 128, 128)], nodebuf.at[r])
            for v in range(8):
                jvals[r, pl.ds(v * 16, 16)] = lax.iota(jnp.int32, 16) + (jb + r * 128 + v * 16)
            pltpu.sync_copy(jvals.at[r], sm_sh.at[nodebuf.at[r]])
        plsc.subcore_barrier()
        pltpu.sync_copy(sm_sh.at[pl.ds(s * 2560, 2560)], mbuf)
        pltpu.sync_copy(mbuf, sm_out.at[pl.ds(s * 2560, 2560)])


def _k2_body(z_ref, w4_ref, b4_ref, q_ref, kv_ref, sk_ref):
    acc = jnp.dot(z_ref[...], w4_ref[...], preferred_element_type=jnp.float32) + b4_ref[...]
    q_ref[...] = acc[:, 0:128]
    kv_ref[...] = acc[:, 128:384]
    sk_ref[...] = acc[:, 384:512]


_k2 = pl.pallas_call(
    _k2_body,
    grid=(NPAD // 1024,),
    in_specs=[
        pl.BlockSpec((1024, 128), lambda i: (i, 0)),
        pl.BlockSpec((128, 512), lambda i: (0, 0)),
        pl.BlockSpec((1, 512), lambda i: (0, 0)),
    ],
    out_specs=[
        pl.BlockSpec((1024, 128), lambda i: (i, 0)),
        pl.BlockSpec((1024, 256), lambda i: (i, 0)),
        pl.BlockSpec((1024, 128), lambda i: (i, 0)),
    ],
    out_shape=[
        jax.ShapeDtypeStruct((NPAD, 128), jnp.float32),
        jax.ShapeDtypeStruct((NPAD, 256), jnp.float32),
        jax.ShapeDtypeStruct((NPAD, 128), jnp.float32),
    ],
)


@functools.partial(
    pl.kernel,
    out_type=[jax.ShapeDtypeStruct((CAP,), jnp.int32)] * 5
    + [jax.ShapeDtypeStruct((NW, 16), jnp.int32)],
    mesh=_mesh,
    compiler_params=pltpu.CompilerParams(needs_layout_passes=False, use_tc_tiling_on_sc=False),
    scratch_types=[
        pltpu.VMEM((NPAD,), jnp.int32),
        pltpu.VMEM((640,), jnp.int32),
        pltpu.VMEM((640,), jnp.int32),
        pltpu.VMEM((640,), jnp.int32),
        pltpu.VMEM((EDGE_W,), jnp.int32),
        pltpu.VMEM((EDGE_W,), jnp.int32),
        pltpu.VMEM((EDGE_W,), jnp.int32),
        pltpu.VMEM((EDGE_W,), jnp.int32),
        pltpu.VMEM((EDGE_W,), jnp.int32),
        pltpu.VMEM((16,), jnp.int32),
        pltpu.SemaphoreType.DMA,
    ],
)
def _k3a(esrc_hbm, edst_hbm, t_hbm, sm_hbm, osrc, odst, oslot, ot, oeid, ocnt,
         smb, srcb, dstb, tb, bsrc, bdst, bslot, bt_, beid, cbuf, sem):
    c = lax.axis_index("c")
    s = lax.axis_index("s")
    w = s * NC + c
    base = w * EDGE_W
    pltpu.sync_copy(sm_hbm, smb)
    iota = lax.iota(jnp.int32, 16)

    def pre(v, carry):
        sl = pl.ds(v * 16, 16)
        z16 = jnp.zeros((16,), jnp.int32)
        bsrc[sl] = z16
        bdst[sl] = z16
        bt_[sl] = z16
        beid[sl] = z16
        bslot[sl] = iota + (TRASH + (v % 8) * 16)
        return carry

    lax.fori_loop(0, CAP_T // 16, pre, 0)

    def batch(i, cnt):
        pltpu.sync_copy(esrc_hbm.at[pl.ds(base + i * 640, 640)], srcb)
        pltpu.sync_copy(edst_hbm.at[pl.ds(base + i * 640, 640)], dstb)
        pltpu.sync_copy(t_hbm.at[pl.ds(base + i * 640, 640)], tb)
        for v in range(40):
            sl = pl.ds(v * 16, 16)
            d = dstb[sl]
            slot = plsc.load_gather(smb, [d])
            m = slot >= 0
            plsc.store_compressed(bslot.at[pl.ds(cnt, 16)], slot, mask=m)
            plsc.store_compressed(bsrc.at[pl.ds(cnt, 16)], srcb[sl], mask=m)
            plsc.store_compressed(bdst.at[pl.ds(cnt, 16)], d, mask=m)
            plsc.store_compressed(bt_.at[pl.ds(cnt, 16)], tb[sl], mask=m)
            plsc.store_compressed(beid.at[pl.ds(cnt, 16)],
                                  iota + (base + i * 640 + v * 16), mask=m)
            cnt = cnt + plsc.all_reduce_population_count(m)[0]
        return cnt

    cnt_f = lax.fori_loop(0, EDGE_W // 640, batch, jnp.int32(0))
    cnt_f = jnp.minimum(cnt_f, CAP_T)
    cbuf[...] = jnp.full((16,), cnt_f, jnp.int32)
    pltpu.sync_copy(cbuf, ocnt.at[w])
    ob = w * CAP_T
    pltpu.sync_copy(bsrc.at[pl.ds(0, CAP_T)], osrc.at[pl.ds(ob, CAP_T)])
    pltpu.sync_copy(bdst.at[pl.ds(0, CAP_T)], odst.at[pl.ds(ob, CAP_T)])
    pltpu.sync_copy(bslot.at[pl.ds(0, CAP_T)], oslot.at[pl.ds(ob, CAP_T)])
    pltpu.sync_copy(bt_.at[pl.ds(0, CAP_T)], ot.at[pl.ds(ob, CAP_T)])
    pltpu.sync_copy(beid.at[pl.ds(0, CAP_T)], oeid.at[pl.ds(ob, CAP_T)])


@functools.partial(
    pl.kernel,
    out_type=jax.ShapeDtypeStruct((CAP,), jnp.float32),
    mesh=_mesh,
    compiler_params=pltpu.CompilerParams(needs_layout_passes=False, use_tc_tiling_on_sc=False),
    scratch_types=[
        pltpu.VMEM((NPAD,), jnp.int32),
        pltpu.VMEM((CAP_T,), jnp.int32),
        pltpu.VMEM((CAP_T,), jnp.int32),
        pltpu.VMEM((CAP_T,), jnp.float32),
        pltpu.SemaphoreType.DMA,
    ],
    name="k3b1_relt",
)
def _k3b1(ssrc, st_, lu_hbm, orelt, lub, src1d, tbuf, reltb, sem):
    c = lax.axis_index("c")
    s = lax.axis_index("s")
    w = s * NC + c
    ob = w * CAP_T
    pltpu.sync_copy(lu_hbm, lub)
    pltpu.sync_copy(ssrc.at[pl.ds(ob, CAP_T)], src1d)
    pltpu.sync_copy(st_.at[pl.ds(ob, CAP_T)], tbuf)

    def rv(v, carry):
        sl = pl.ds(v * 16, 16)
        s16 = src1d[sl]
        lu16 = plsc.load_gather(lub, [s16])
        reltb[sl] = (lu16 - tbuf[sl]).astype(jnp.float32)
        return carry

    lax.fori_loop(0, CAP_T // 16, rv, 0)
    pltpu.sync_copy(reltb, orelt.at[pl.ds(ob, CAP_T)])


_GCH = 32
_NCH = CAP_T // _GCH
_DEP = 6


@functools.partial(
    pl.kernel,
    out_type=[
        jax.ShapeDtypeStruct((CAP, MSG_DIM), jnp.float32),
        jax.ShapeDtypeStruct((CAP, 128), jnp.float32),
        jax.ShapeDtypeStruct((CAP, 256), jnp.float32),
    ],
    mesh=_mesh,
    compiler_params=pltpu.CompilerParams(needs_layout_passes=False, use_tc_tiling_on_sc=False),
    scratch_types=[
        pltpu.VMEM((_NCH, _GCH), jnp.int32),
        pltpu.VMEM((_NCH, _GCH), jnp.int32),
        pltpu.VMEM((_NCH, _GCH), jnp.int32),
        pltpu.VMEM((16,), jnp.int32),
    ] + [pltpu.VMEM((_GCH, MSG_DIM), jnp.float32)] * _DEP
      + [pltpu.VMEM((_GCH, 128), jnp.float32)] * _DEP
      + [pltpu.VMEM((_GCH, 256), jnp.float32)] * _DEP
      + [pltpu.SemaphoreType.DMA, pltpu.SemaphoreType.DMA],
    name="k3b2_gather",
)
def _k3b2(ssrc2, sdst2, seid2, q_hbm, kv_hbm, msg_hbm, cnt_hbm,
          omsg, oq, okv,
          srcb, dstb, eidb, cbuf,
          m0, m1, m2, m3, m4, m5,
          q0, q1, q2, q3, q4, q5,
          k0, k1, k2, k3, k4, k5, gsem, osem):
    c = lax.axis_index("c")
    s = lax.axis_index("s")
    w = s * NC + c
    ob = w * CAP_T
    rb = w * _NCH
    pltpu.sync_copy(ssrc2.at[pl.ds(rb, _NCH), :], srcb)
    pltpu.sync_copy(sdst2.at[pl.ds(rb, _NCH), :], dstb)
    pltpu.sync_copy(seid2.at[pl.ds(rb, _NCH), :], eidb)
    pltpu.sync_copy(cnt_hbm.at[w], cbuf)
    nch = (cbuf[...][0] + (_GCH - 1)) // _GCH
    msgt = (m0, m1, m2, m3, m4, m5)
    qt = (q0, q1, q2, q3, q4, q5)
    kvt = (k0, k1, k2, k3, k4, k5)

    def issue_g(r):
        bi = r % _DEP
        pltpu.async_copy(msg_hbm.at[eidb.at[r]], msgt[bi], gsem)
        pltpu.async_copy(q_hbm.at[dstb.at[r]], qt[bi], gsem)
        pltpu.async_copy(kv_hbm.at[srcb.at[r]], kvt[bi], gsem)

    def drain_g(bi):
        pltpu.make_async_copy(msg_hbm.at[pl.ds(0, _GCH), :], msgt[bi], gsem).wait()
        pltpu.make_async_copy(q_hbm.at[pl.ds(0, _GCH), :], qt[bi], gsem).wait()
        pltpu.make_async_copy(kv_hbm.at[pl.ds(0, _GCH), :], kvt[bi], gsem).wait()

    def issue_o(r):
        bi = r % _DEP
        sl = pl.ds(ob + r * _GCH, _GCH)
        pltpu.async_copy(msgt[bi], omsg.at[sl, :], osem)
        pltpu.async_copy(qt[bi], oq.at[sl, :], osem)
        pltpu.async_copy(kvt[bi], okv.at[sl, :], osem)

    def drain_o(bi):
        pltpu.make_async_copy(msgt[bi], omsg.at[pl.ds(0, _GCH), :], osem).wait()
        pltpu.make_async_copy(qt[bi], oq.at[pl.ds(0, _GCH), :], osem).wait()
        pltpu.make_async_copy(kvt[bi], okv.at[pl.ds(0, _GCH), :], osem).wait()

    issued_o = set()
    drained_o = set()
    for r in range(_NCH + 4):
        x = r - _DEP
        if x in issued_o and x not in drained_o:
            drained_o.add(x)

            @pl.when(x < nch)
            def _(x=x):
                drain_o(x % _DEP)

        if r < _NCH:
            @pl.when(r < nch)
            def _(r=r):
                issue_g(r)

        y = r - 4
        if 0 <= y < _NCH:
            issued_o.add(y)

            @pl.when(y < nch)
            def _(y=y):
                drain_g(y % _DEP)
                issue_o(y)

    for x in range(_NCH):
        if x in issued_o and x not in drained_o:
            @pl.when(x < nch)
            def _(x=x):
                drain_o(x % _DEP)


def _k4_body(relt_ref, msg_ref, kvs_ref, qs_ref, wt_ref, btb_ref, wet_ref, wem_ref, p_ref):
    relt = relt_ref[...]
    enc = jnp.cos(relt * wt_ref[...] + btb_ref[...])
    ev = jnp.dot(enc, wet_ref[...], preferred_element_type=jnp.float32)
    ev = ev + jnp.dot(msg_ref[...], wem_ref[...], preferred_element_type=jnp.float32)
    kvs = kvs_ref[...]
    ke = kvs[:, 0:128] + ev
    ve = kvs[:, 128:256] + ev
    prod = qs_ref[...] * ke
    a0 = jnp.sum(prod[:, 0:64], axis=1, keepdims=True) * 0.125
    a1 = jnp.sum(prod[:, 64:128], axis=1, keepdims=True) * 0.125
    e0 = jnp.exp(a0)
    e1 = jnp.exp(a1)
    vex = ve * jnp.concatenate(
        [jnp.broadcast_to(e0, (1024, 64)), jnp.broadcast_to(e1, (1024, 64))], axis=1)
    lane = lax.broadcasted_iota(jnp.int32, (1024, 16), 1)
    extra = jnp.where(lane == 0, e0, jnp.where(lane == 1, e1, jnp.float32(0)))
    p_ref[...] = jnp.concatenate([vex, extra], axis=1)


_k4 = pl.pallas_call(
    _k4_body,
    grid=(CAP // 1024,),
    in_specs=[
        pl.BlockSpec((1024, 1), lambda i: (i, 0)),
        pl.BlockSpec((1024, MSG_DIM), lambda i: (i, 0)),
        pl.BlockSpec((1024, 256), lambda i: (i, 0)),
        pl.BlockSpec((1024, 128), lambda i: (i, 0)),
        pl.BlockSpec((1, TIME_DIM), lambda i: (0, 0)),
        pl.BlockSpec((1, TIME_DIM), lambda i: (0, 0)),
        pl.BlockSpec((TIME_DIM, 128), lambda i: (0, 0)),
        pl.BlockSpec((MSG_DIM, 128), lambda i: (0, 0)),
    ],
    out_specs=pl.BlockSpec((1024, PAYW), lambda i: (i, 0)),
    out_shape=jax.ShapeDtypeStruct((CAP, PAYW), jnp.float32),
)


@functools.partial(
    pl.kernel,
    out_type=[
        jax.ShapeDtypeStruct((NSLOT, PAYW), jnp.float32),
        jax.ShapeDtypeStruct((NSLOT, PAYW), jnp.float32),
    ],
    mesh=_mesh,
    compiler_params=pltpu.CompilerParams(needs_layout_passes=False, use_tc_tiling_on_sc=False),
    scratch_types=[
        pltpu.VMEM((24, 128), jnp.int32),
        pltpu.VMEM((128, PAYW), jnp.float32),
        pltpu.VMEM((132, PAYW), jnp.float32),
        pltpu.VMEM((16,), jnp.int32),
        pltpu.VMEM_SHARED((NSLOT, PAYW), jnp.float32),
        pltpu.SemaphoreType.DMA,
    ],
)
def _k5(p_hbm, slot_hbm, cnt_hbm, tab0_out, tab1_out, slotb, pbuf, stage, cbuf5, tab_sh, sem):
    c = lax.axis_index("c")
    s = lax.axis_index("s")
    w = s * NC + c
    ob = w * CAP_T
    for r in range(132):
        for v in range(PAYW // 16):
            stage[r, pl.ds(v * 16, 16)] = jnp.zeros((16,), jnp.float32)

    def zs(jj, carry):
        pltpu.sync_copy(stage, tab_sh.at[pl.ds(s * 528 + jj * 132, 132), :])
        return carry

    lax.fori_loop(0, 4, zs, 0)
    plsc.subcore_barrier()

    def r24(r, carry):
        pltpu.sync_copy(slot_hbm.at[pl.ds(ob + r * 128, 128)], slotb.at[r])
        pltpu.sync_copy(p_hbm.at[pl.ds(ob + r * 128, 128), :], pbuf)
        pltpu.sync_copy(pbuf, tab_sh.at[slotb.at[r]], add=True)
        return carry

    pltpu.sync_copy(cnt_hbm.at[w], cbuf5)
    nch5 = (cbuf5[...][0] + 127) // 128
    lax.fori_loop(0, nch5, r24, 0)
    plsc.subcore_barrier()

    def dmp(jj, carry):
        pltpu.sync_copy(tab_sh.at[pl.ds(s * 528 + jj * 132, 132), :], stage)

        @pl.when(c == 0)
        def _():
            pltpu.sync_copy(stage, tab0_out.at[pl.ds(s * 528 + jj * 132, 132), :])

        @pl.when(c == 1)
        def _():
            pltpu.sync_copy(stage, tab1_out.at[pl.ds(s * 528 + jj * 132, 132), :])

        return carry

    lax.fori_loop(0, 4, dmp, 0)


@functools.partial(
    pl.kernel,
    out_type=[
        jax.ShapeDtypeStruct((2 * B, PAYW), jnp.float32),
        jax.ShapeDtypeStruct((2 * B, PAYW), jnp.float32),
        jax.ShapeDtypeStruct((2 * B, 128), jnp.float32),
    ],
    mesh=_mesh,
    compiler_params=pltpu.CompilerParams(needs_layout_passes=False, use_tc_tiling_on_sc=False),
    scratch_types=[
        pltpu.VMEM((NPAD,), jnp.int32),
        pltpu.VMEM((2, 128), jnp.int32),
        pltpu.VMEM((2, 128), jnp.int32),
        pltpu.VMEM((128, PAYW), jnp.float32),
        pltpu.VMEM((128, 128), jnp.float32),
        pltpu.SemaphoreType.DMA,
    ],
)
def _k6(sm_hbm, srcdst_hbm, tab0_hbm, tab1_hbm, skip_hbm, g0, g1, sk,
        smb, nb, sb, gt, skt, sem):
    c = lax.axis_index("c")
    s = lax.axis_index("s")
    w = s * NC + c
    ob = w * (2 * B // NW)
    pltpu.sync_copy(sm_hbm, smb)
    for r in range(2):
        pltpu.sync_copy(srcdst_hbm.at[pl.ds(ob + r * 128, 128)], nb.at[r])
        for v in range(8):
            n16 = nb[r, pl.ds(v * 16, 16)]
            sb[r, pl.ds(v * 16, 16)] = plsc.load_gather(smb, [n16])
        pltpu.async_copy(tab0_hbm.at[sb.at[r]], gt, sem).wait()
        pltpu.sync_copy(gt, g0.at[pl.ds(ob + r * 128, 128), :])
        pltpu.async_copy(tab1_hbm.at[sb.at[r]], gt, sem).wait()
        pltpu.sync_copy(gt, g1.at[pl.ds(ob + r * 128, 128), :])
        pltpu.async_copy(skip_hbm.at[nb.at[r]], skt, sem).wait()
        pltpu.sync_copy(skt, sk.at[pl.ds(ob + r * 128, 128), :])


_BLK7 = 512


def _k7_body(g0s, g1s, sks, g0d, g1d, skd, wsrc_ref, wdst_ref, bh_ref, wout_ref,
             bout_ref, y_ref):
    def node_out(a, b, sk):
        num = a[:, 0:128] + b[:, 0:128]
        d0 = a[:, 128:129] + b[:, 128:129]
        d1 = a[:, 129:130] + b[:, 129:130]
        den = jnp.concatenate(
            [jnp.broadcast_to(d0, (_BLK7, 64)), jnp.broadcast_to(d1, (_BLK7, 64))],
            axis=1)
        return num / (den + 1e-16) + sk

    os_ = node_out(g0s[...], g1s[...], sks[...])
    od_ = node_out(g0d[...], g1d[...], skd[...])
    h = os_ @ wsrc_ref[...] + od_ @ wdst_ref[...] + bh_ref[...]
    h = jnp.maximum(h, 0.0)
    y_ref[...] = h @ wout_ref[...] + bout_ref[...]


_k7 = pl.pallas_call(
    _k7_body,
    grid=(B // _BLK7,),
    in_specs=[
        pl.BlockSpec((_BLK7, PAYW), lambda i: (i, 0)),
        pl.BlockSpec((_BLK7, PAYW), lambda i: (i, 0)),
        pl.BlockSpec((_BLK7, 128), lambda i: (i, 0)),
        pl.BlockSpec((_BLK7, PAYW), lambda i: (i, 0)),
        pl.BlockSpec((_BLK7, PAYW), lambda i: (i, 0)),
        pl.BlockSpec((_BLK7, 128), lambda i: (i, 0)),
        pl.BlockSpec((128, 128), lambda i: (0, 0)),
        pl.BlockSpec((128, 128), lambda i: (0, 0)),
        pl.BlockSpec((1, 128), lambda i: (0, 0)),
        pl.BlockSpec((128, OUT_CH), lambda i: (0, 0)),
        pl.BlockSpec((1, OUT_CH), lambda i: (0, 0)),
    ],
    out_specs=pl.BlockSpec((_BLK7, OUT_CH), lambda i: (i, 0)),
    out_shape=jax.ShapeDtypeStruct((B, OUT_CH), jnp.float32),
)


def kernel(n_id, edge_index, t, msg, src, dst, memory, last_update, Wt, bt, Wq, bq, Wk, bk, Wv, bv, We, Wskip, bskip, Wsrc, Wdst, bh, Wout, bout):
    nid_p = jnp.concatenate([n_id, jnp.zeros((NPAD - N_BATCH,), jnp.int32)])
    srcdst = jnp.concatenate([src, dst])
    z, lu, slotmap = _k1(memory, last_update, nid_p, srcdst)

    w4 = jnp.concatenate([Wq, Wk, Wv, Wskip], axis=1)
    b4 = jnp.concatenate([bq, bk, bv, bskip])[None, :]
    q, kv, skip = _k2(z, w4, b4)

    epad = jnp.full((EPAD - N_EDGES,), N_BATCH, jnp.int32)
    esrc = jnp.concatenate([edge_index[0], epad])
    edst = jnp.concatenate([edge_index[1], epad])
    tp = jnp.concatenate([t, jnp.zeros((EPAD - N_EDGES,), jnp.int32)])
    ssrc, sdst, sslot, st_, seid, cnts = _k3a(esrc, edst, tp, slotmap)

    relt = _k3b1(ssrc, st_, lu)
    ssrc2 = ssrc.reshape(CAP // _GCH, _GCH)
    sdst2 = sdst.reshape(CAP // _GCH, _GCH)
    seid2 = seid.reshape(CAP // _GCH, _GCH)
    msgs, qs, kvs = _k3b2(ssrc2, sdst2, seid2, q, kv, msg, cnts)

    p = _k4(relt[:, None], msgs, kvs, qs, Wt, bt[None, :], We[:TIME_DIM], We[TIME_DIM:])

    tab0, tab1 = _k5(p, sslot, cnts)

    g0, g1, sk = _k6(slotmap, srcdst, tab0, tab1, skip)

    y = _k7(g0[:B], g1[:B], sk[:B], g0[B:], g1[B:], sk[B:],
            Wsrc, Wdst, bh[None, :], Wout, bout[None, :])
    return y


# speedup vs baseline: 2.6519x; 1.0233x over previous
import functools

import jax
import jax.numpy as jnp
from jax import lax
from jax.experimental import pallas as pl
from jax.experimental.pallas import tpu as pltpu
from jax.experimental.pallas import tpu_sc as plsc

NUM_NODES = 100000
MEM_DIM = 128
TIME_DIM = 16
MSG_DIM = 16
EMBED_DIM = 128
HEADS = 2
DH = EMBED_DIM // HEADS
OUT_CH = 100
N_BATCH = 40000
N_EDGES = 400000
B = 4096

NC = 2
NS = 16
NW = NC * NS
NPAD = 40960
EPAD = 409600
NODE_W = NPAD // NW
EDGE_W = EPAD // NW
CAP_T = 3072
CAP = CAP_T * NW
NSLOT = 8448
TRASH = 8192
PAYW = 144

_mesh = plsc.VectorSubcoreMesh(core_axis_name="c", subcore_axis_name="s")


@functools.partial(
    pl.kernel,
    out_type=[
        jax.ShapeDtypeStruct((NPAD, MEM_DIM), jnp.float32),
        jax.ShapeDtypeStruct((NPAD,), jnp.int32),
        jax.ShapeDtypeStruct((NPAD,), jnp.int32),
    ],
    mesh=_mesh,
    compiler_params=pltpu.CompilerParams(needs_layout_passes=False, use_tc_tiling_on_sc=False),
    scratch_types=[
        pltpu.VMEM((NODE_W,), jnp.int32),
        pltpu.VMEM((128, MEM_DIM), jnp.float32),
        pltpu.VMEM((128, MEM_DIM), jnp.float32),
        pltpu.VMEM((128,), jnp.int32),
        pltpu.VMEM((128,), jnp.int32),
        pltpu.VMEM((2560,), jnp.int32),
        pltpu.VMEM((4, 128), jnp.int32),
        pltpu.VMEM((4, 128), jnp.int32),
        pltpu.VMEM_SHARED((NPAD,), jnp.int32),
        pltpu.SemaphoreType.DMA,
    ],
)
def _k1(mem_hbm, lu_hbm, nid_hbm, srcdst_hbm, z_out, lu_out, sm_out,
        idbuf, zbuf0, zbuf1, lubuf0, lubuf1, mbuf, nodebuf, jvals, sm_sh, sem):
    c = lax.axis_index("c")
    s = lax.axis_index("s")
    w = s * NC + c
    base = w * NODE_W
    pltpu.sync_copy(nid_hbm.at[pl.ds(base, NODE_W)], idbuf)
    zb = (zbuf0, zbuf1)
    lb = (lubuf0, lubuf1)
    _NKC = NODE_W // 128

    def issue_g1(i):
        idx = idbuf.at[pl.ds(i * 128, 128)]
        return (pltpu.async_copy(mem_hbm.at[idx], zb[i % 2], sem),
                pltpu.async_copy(lu_hbm.at[idx], lb[i % 2], sem))

    def issue_o1(i):
        sl = pl.ds(base + i * 128, 128)
        return (pltpu.async_copy(zb[i % 2], z_out.at[sl, :], sem),
                pltpu.async_copy(lb[i % 2], lu_out.at[sl], sem))

    g1 = {}
    o1 = {}
    for r in range(_NKC + 1):
        if (r - 2) in o1:
            for d in o1.pop(r - 2):
                d.wait()
        if r < _NKC:
            g1[r] = issue_g1(r)
        if (r - 1) in g1:
            for d in g1.pop(r - 1):
                d.wait()
            o1[r - 1] = issue_o1(r - 1)
    for rr in sorted(o1):
        for d in o1[rr]:
            d.wait()

    @pl.when(c == 0)
    def _():
        def pre(v, carry):
            mbuf[pl.ds(v * 16, 16)] = jnp.full((16,), -1, jnp.int32)
            return carry

        lax.fori_loop(0, 2560 // 16, pre, 0)
        pltpu.sync_copy(mbuf, sm_sh.at[pl.ds(s * 2560, 2560)])
        plsc.subcore_barrier()
        jb = s * 512
        for r in range(4):
            pltpu.sync_copy(srcdst_hbm.at[pl.ds(jb + r * 128, 128)], nodebuf.at[r])
            for v in range(8):
                jvals[r, pl.ds(v * 16, 16)] = lax.iota(jnp.int32, 16) + (jb + r * 128 + v * 16)
            pltpu.sync_copy(jvals.at[r], sm_sh.at[nodebuf.at[r]])
        plsc.subcore_barrier()
        pltpu.sync_copy(sm_sh.at[pl.ds(s * 2560, 2560)], mbuf)
        pltpu.sync_copy(mbuf, sm_out.at[pl.ds(s * 2560, 2560)])


def _k2_body(z_ref, w4_ref, b4_ref, q_ref, kv_ref, sk_ref):
    acc = jnp.dot(z_ref[...], w4_ref[...], preferred_element_type=jnp.float32) + b4_ref[...]
    q_ref[...] = acc[:, 0:128]
    kv_ref[...] = acc[:, 128:384]
    sk_ref[...] = acc[:, 384:512]


_k2 = pl.pallas_call(
    _k2_body,
    grid=(NPAD // 1024,),
    in_specs=[
        pl.BlockSpec((1024, 128), lambda i: (i, 0)),
        pl.BlockSpec((128, 512), lambda i: (0, 0)),
        pl.BlockSpec((1, 512), lambda i: (0, 0)),
    ],
    out_specs=[
        pl.BlockSpec((1024, 128), lambda i: (i, 0)),
        pl.BlockSpec((1024, 256), lambda i: (i, 0)),
        pl.BlockSpec((1024, 128), lambda i: (i, 0)),
    ],
    out_shape=[
        jax.ShapeDtypeStruct((NPAD, 128), jnp.float32),
        jax.ShapeDtypeStruct((NPAD, 256), jnp.float32),
        jax.ShapeDtypeStruct((NPAD, 128), jnp.float32),
    ],
)


@functools.partial(
    pl.kernel,
    out_type=[jax.ShapeDtypeStruct((CAP,), jnp.int32)] * 5
    + [jax.ShapeDtypeStruct((NW, 16), jnp.int32)],
    mesh=_mesh,
    compiler_params=pltpu.CompilerParams(needs_layout_passes=False, use_tc_tiling_on_sc=False),
    scratch_types=[
        pltpu.VMEM((NPAD,), jnp.int32),
        pltpu.VMEM((640,), jnp.int32),
        pltpu.VMEM((640,), jnp.int32),
        pltpu.VMEM((640,), jnp.int32),
        pltpu.VMEM((EDGE_W,), jnp.int32),
        pltpu.VMEM((EDGE_W,), jnp.int32),
        pltpu.VMEM((EDGE_W,), jnp.int32),
        pltpu.VMEM((EDGE_W,), jnp.int32),
        pltpu.VMEM((EDGE_W,), jnp.int32),
        pltpu.VMEM((16,), jnp.int32),
        pltpu.SemaphoreType.DMA,
    ],
)
def _k3a(esrc_hbm, edst_hbm, t_hbm, sm_hbm, osrc, odst, oslot, ot, oeid, ocnt,
         smb, srcb, dstb, tb, bsrc, bdst, bslot, bt_, beid, cbuf, sem):
    c = lax.axis_index("c")
    s = lax.axis_index("s")
    w = s * NC + c
    base = w * EDGE_W
    pltpu.sync_copy(sm_hbm, smb)
    iota = lax.iota(jnp.int32, 16)

    def pre(v, carry):
        sl = pl.ds(v * 16, 16)
        z16 = jnp.zeros((16,), jnp.int32)
        bsrc[sl] = z16
        bdst[sl] = z16
        bt_[sl] = z16
        beid[sl] = z16
        bslot[sl] = iota + (TRASH + (v % 8) * 16)
        return carry

    lax.fori_loop(0, CAP_T // 16, pre, 0)

    def batch(i, cnt):
        pltpu.sync_copy(esrc_hbm.at[pl.ds(base + i * 640, 640)], srcb)
        pltpu.sync_copy(edst_hbm.at[pl.ds(base + i * 640, 640)], dstb)
        pltpu.sync_copy(t_hbm.at[pl.ds(base + i * 640, 640)], tb)
        for v in range(40):
            sl = pl.ds(v * 16, 16)
            d = dstb[sl]
            slot = plsc.load_gather(smb, [d])
            m = slot >= 0
            plsc.store_compressed(bslot.at[pl.ds(cnt, 16)], slot, mask=m)
            plsc.store_compressed(bsrc.at[pl.ds(cnt, 16)], srcb[sl], mask=m)
            plsc.store_compressed(bdst.at[pl.ds(cnt, 16)], d, mask=m)
            plsc.store_compressed(bt_.at[pl.ds(cnt, 16)], tb[sl], mask=m)
            plsc.store_compressed(beid.at[pl.ds(cnt, 16)],
                                  iota + (base + i * 640 + v * 16), mask=m)
            cnt = cnt + plsc.all_reduce_population_count(m)[0]
        return cnt

    cnt_f = lax.fori_loop(0, EDGE_W // 640, batch, jnp.int32(0))
    cnt_f = jnp.minimum(cnt_f, CAP_T)
    cbuf[...] = jnp.full((16,), cnt_f, jnp.int32)
    pltpu.sync_copy(cbuf, ocnt.at[w])
    ob = w * CAP_T
    pltpu.sync_copy(bsrc.at[pl.ds(0, CAP_T)], osrc.at[pl.ds(ob, CAP_T)])
    pltpu.sync_copy(bdst.at[pl.ds(0, CAP_T)], odst.at[pl.ds(ob, CAP_T)])
    pltpu.sync_copy(bslot.at[pl.ds(0, CAP_T)], oslot.at[pl.ds(ob, CAP_T)])
    pltpu.sync_copy(bt_.at[pl.ds(0, CAP_T)], ot.at[pl.ds(ob, CAP_T)])
    pltpu.sync_copy(beid.at[pl.ds(0, CAP_T)], oeid.at[pl.ds(ob, CAP_T)])


@functools.partial(
    pl.kernel,
    out_type=jax.ShapeDtypeStruct((CAP,), jnp.float32),
    mesh=_mesh,
    compiler_params=pltpu.CompilerParams(needs_layout_passes=False, use_tc_tiling_on_sc=False),
    scratch_types=[
        pltpu.VMEM((NPAD,), jnp.int32),
        pltpu.VMEM((CAP_T,), jnp.int32),
        pltpu.VMEM((CAP_T,), jnp.int32),
        pltpu.VMEM((CAP_T,), jnp.float32),
        pltpu.SemaphoreType.DMA,
    ],
    name="k3b1_relt",
)
def _k3b1(ssrc, st_, lu_hbm, orelt, lub, src1d, tbuf, reltb, sem):
    c = lax.axis_index("c")
    s = lax.axis_index("s")
    w = s * NC + c
    ob = w * CAP_T
    pltpu.sync_copy(lu_hbm, lub)
    pltpu.sync_copy(ssrc.at[pl.ds(ob, CAP_T)], src1d)
    pltpu.sync_copy(st_.at[pl.ds(ob, CAP_T)], tbuf)

    def rv(v, carry):
        sl = pl.ds(v * 16, 16)
        s16 = src1d[sl]
        lu16 = plsc.load_gather(lub, [s16])
        reltb[sl] = (lu16 - tbuf[sl]).astype(jnp.float32)
        return carry

    lax.fori_loop(0, CAP_T // 16, rv, 0)
    pltpu.sync_copy(reltb, orelt.at[pl.ds(ob, CAP_T)])


_GCH = 32
_NCH = CAP_T // _GCH
_DEP = 6


@functools.partial(
    pl.kernel,
    out_type=[
        jax.ShapeDtypeStruct((CAP, MSG_DIM), jnp.float32),
        jax.ShapeDtypeStruct((CAP, 128), jnp.float32),
        jax.ShapeDtypeStruct((CAP, 256), jnp.float32),
    ],
    mesh=_mesh,
    compiler_params=pltpu.CompilerParams(needs_layout_passes=False, use_tc_tiling_on_sc=False),
    scratch_types=[
        pltpu.VMEM((_NCH, _GCH), jnp.int32),
        pltpu.VMEM((_NCH, _GCH), jnp.int32),
        pltpu.VMEM((_NCH, _GCH), jnp.int32),
        pltpu.VMEM((16,), jnp.int32),
    ] + [pltpu.VMEM((_GCH, MSG_DIM), jnp.float32)] * _DEP
      + [pltpu.VMEM((_GCH, 128), jnp.float32)] * _DEP
      + [pltpu.VMEM((_GCH, 256), jnp.float32)] * _DEP
      + [pltpu.SemaphoreType.DMA, pltpu.SemaphoreType.DMA],
    name="k3b2_gather",
)
def _k3b2(ssrc2, sdst2, seid2, q_hbm, kv_hbm, msg_hbm, cnt_hbm,
          omsg, oq, okv,
          srcb, dstb, eidb, cbuf,
          m0, m1, m2, m3, m4, m5,
          q0, q1, q2, q3, q4, q5,
          k0, k1, k2, k3, k4, k5, gsem, osem):
    c = lax.axis_index("c")
    s = lax.axis_index("s")
    w = s * NC + c
    ob = w * CAP_T
    rb = w * _NCH
    pltpu.sync_copy(ssrc2.at[pl.ds(rb, _NCH), :], srcb)
    pltpu.sync_copy(sdst2.at[pl.ds(rb, _NCH), :], dstb)
    pltpu.sync_copy(seid2.at[pl.ds(rb, _NCH), :], eidb)
    pltpu.sync_copy(cnt_hbm.at[w], cbuf)
    nch = (cbuf[...][0] + (_GCH - 1)) // _GCH
    msgt = (m0, m1, m2, m3, m4, m5)
    qt = (q0, q1, q2, q3, q4, q5)
    kvt = (k0, k1, k2, k3, k4, k5)

    def issue_g(r):
        bi = r % _DEP
        pltpu.async_copy(msg_hbm.at[eidb.at[r]], msgt[bi], gsem)
        pltpu.async_copy(q_hbm.at[dstb.at[r]], qt[bi], gsem)
        pltpu.async_copy(kv_hbm.at[srcb.at[r]], kvt[bi], gsem)

    def drain_g(bi):
        pltpu.make_async_copy(msg_hbm.at[pl.ds(0, _GCH), :], msgt[bi], gsem).wait()
        pltpu.make_async_copy(q_hbm.at[pl.ds(0, _GCH), :], qt[bi], gsem).wait()
        pltpu.make_async_copy(kv_hbm.at[pl.ds(0, _GCH), :], kvt[bi], gsem).wait()

    def issue_o(r):
        bi = r % _DEP
        sl = pl.ds(ob + r * _GCH, _GCH)
        pltpu.async_copy(msgt[bi], omsg.at[sl, :], osem)
        pltpu.async_copy(qt[bi], oq.at[sl, :], osem)
        pltpu.async_copy(kvt[bi], okv.at[sl, :], osem)

    def drain_o(bi):
        pltpu.make_async_copy(msgt[bi], omsg.at[pl.ds(0, _GCH), :], osem).wait()
        pltpu.make_async_copy(qt[bi], oq.at[pl.ds(0, _GCH), :], osem).wait()
        pltpu.make_async_copy(kvt[bi], okv.at[pl.ds(0, _GCH), :], osem).wait()

    issued_o = set()
    drained_o = set()
    for r in range(_NCH + 4):
        x = r - _DEP
        if x in issued_o and x not in drained_o:
            drained_o.add(x)

            @pl.when(x < nch)
            def _(x=x):
                drain_o(x % _DEP)

        if r < _NCH:
            @pl.when(r < nch)
            def _(r=r):
                issue_g(r)

        y = r - 4
        if 0 <= y < _NCH:
            issued_o.add(y)

            @pl.when(y < nch)
            def _(y=y):
                drain_g(y % _DEP)
                issue_o(y)

    for x in range(_NCH):
        if x in issued_o and x not in drained_o:
            @pl.when(x < nch)
            def _(x=x):
                drain_o(x % _DEP)


def _k4_body(relt_ref, msg_ref, kvs_ref, qs_ref, wt_ref, btb_ref, wet_ref, wem_ref, p_ref):
    relt = relt_ref[...]
    enc = jnp.cos(relt * wt_ref[...] + btb_ref[...])
    ev = jnp.dot(enc, wet_ref[...], preferred_element_type=jnp.float32)
    ev = ev + jnp.dot(msg_ref[...], wem_ref[...], preferred_element_type=jnp.float32)
    kvs = kvs_ref[...]
    ke = kvs[:, 0:128] + ev
    ve = kvs[:, 128:256] + ev
    prod = qs_ref[...] * ke
    a0 = jnp.sum(prod[:, 0:64], axis=1, keepdims=True) * 0.125
    a1 = jnp.sum(prod[:, 64:128], axis=1, keepdims=True) * 0.125
    e0 = jnp.exp(a0)
    e1 = jnp.exp(a1)
    vex = ve * jnp.concatenate(
        [jnp.broadcast_to(e0, (1024, 64)), jnp.broadcast_to(e1, (1024, 64))], axis=1)
    lane = lax.broadcasted_iota(jnp.int32, (1024, 16), 1)
    extra = jnp.where(lane == 0, e0, jnp.where(lane == 1, e1, jnp.float32(0)))
    p_ref[...] = jnp.concatenate([vex, extra], axis=1)


_k4 = pl.pallas_call(
    _k4_body,
    grid=(CAP // 1024,),
    in_specs=[
        pl.BlockSpec((1024, 1), lambda i: (i, 0)),
        pl.BlockSpec((1024, MSG_DIM), lambda i: (i, 0)),
        pl.BlockSpec((1024, 256), lambda i: (i, 0)),
        pl.BlockSpec((1024, 128), lambda i: (i, 0)),
        pl.BlockSpec((1, TIME_DIM), lambda i: (0, 0)),
        pl.BlockSpec((1, TIME_DIM), lambda i: (0, 0)),
        pl.BlockSpec((TIME_DIM, 128), lambda i: (0, 0)),
        pl.BlockSpec((MSG_DIM, 128), lambda i: (0, 0)),
    ],
    out_specs=pl.BlockSpec((1024, PAYW), lambda i: (i, 0)),
    out_shape=jax.ShapeDtypeStruct((CAP, PAYW), jnp.float32),
)


@functools.partial(
    pl.kernel,
    out_type=[
        jax.ShapeDtypeStruct((NSLOT, PAYW), jnp.float32),
        jax.ShapeDtypeStruct((NSLOT, PAYW), jnp.float32),
    ],
    mesh=_mesh,
    compiler_params=pltpu.CompilerParams(needs_layout_passes=False, use_tc_tiling_on_sc=False),
    scratch_types=[
        pltpu.VMEM((24, 128), jnp.int32),
        pltpu.VMEM((128, PAYW), jnp.float32),
        pltpu.VMEM((132, PAYW), jnp.float32),
        pltpu.VMEM((16,), jnp.int32),
        pltpu.VMEM_SHARED((NSLOT, PAYW), jnp.float32),
        pltpu.SemaphoreType.DMA,
    ],
)
def _k5(p_hbm, slot_hbm, cnt_hbm, tab0_out, tab1_out, slotb, pbuf, stage, cbuf5, tab_sh, sem):
    c = lax.axis_index("c")
    s = lax.axis_index("s")
    w = s * NC + c
    ob = w * CAP_T
    for r in range(132):
        for v in range(PAYW // 16):
            stage[r, pl.ds(v * 16, 16)] = jnp.zeros((16,), jnp.float32)

    def zs(jj, carry):
        pltpu.sync_copy(stage, tab_sh.at[pl.ds(s * 528 + jj * 132, 132), :])
        return carry

    lax.fori_loop(0, 4, zs, 0)
    plsc.subcore_barrier()

    def r24(r, carry):
        pltpu.sync_copy(slot_hbm.at[pl.ds(ob + r * 128, 128)], slotb.at[r])
        pltpu.sync_copy(p_hbm.at[pl.ds(ob + r * 128, 128), :], pbuf)
        pltpu.sync_copy(pbuf, tab_sh.at[slotb.at[r]], add=True)
        return carry

    pltpu.sync_copy(cnt_hbm.at[w], cbuf5)
    nch5 = (cbuf5[...][0] + 127) // 128
    lax.fori_loop(0, nch5, r24, 0)
    plsc.subcore_barrier()

    def dmp(jj, carry):
        pltpu.sync_copy(tab_sh.at[pl.ds(s * 528 + jj * 132, 132), :], stage)

        @pl.when(c == 0)
        def _():
            pltpu.sync_copy(stage, tab0_out.at[pl.ds(s * 528 + jj * 132, 132), :])

        @pl.when(c == 1)
        def _():
            pltpu.sync_copy(stage, tab1_out.at[pl.ds(s * 528 + jj * 132, 132), :])

        return carry

    lax.fori_loop(0, 4, dmp, 0)


@functools.partial(
    pl.kernel,
    out_type=[
        jax.ShapeDtypeStruct((2 * B, PAYW), jnp.float32),
        jax.ShapeDtypeStruct((2 * B, PAYW), jnp.float32),
        jax.ShapeDtypeStruct((2 * B, 128), jnp.float32),
    ],
    mesh=_mesh,
    compiler_params=pltpu.CompilerParams(needs_layout_passes=False, use_tc_tiling_on_sc=False),
    scratch_types=[
        pltpu.VMEM((NPAD,), jnp.int32),
        pltpu.VMEM((2, 128), jnp.int32),
        pltpu.VMEM((2, 128), jnp.int32),
        pltpu.VMEM((128, PAYW), jnp.float32),
        pltpu.VMEM((128, PAYW), jnp.float32),
        pltpu.VMEM((128, 128), jnp.float32),
        pltpu.SemaphoreType.DMA,
        pltpu.SemaphoreType.DMA,
    ],
)
def _k6(sm_hbm, srcdst_hbm, tab0_hbm, tab1_hbm, skip_hbm, g0, g1, sk,
        smb, nb, sb, gt0, gt1, skt, sem, osem):
    c = lax.axis_index("c")
    s = lax.axis_index("s")
    w = s * NC + c
    ob = w * (2 * B // NW)
    pltpu.sync_copy(sm_hbm, smb)
    pltpu.sync_copy(srcdst_hbm.at[pl.ds(ob, 128)], nb.at[0])
    pltpu.sync_copy(srcdst_hbm.at[pl.ds(ob + 128, 128)], nb.at[1])
    for r in range(2):
        for v in range(8):
            n16 = nb[r, pl.ds(v * 16, 16)]
            sb[r, pl.ds(v * 16, 16)] = plsc.load_gather(smb, [n16])
    for r in range(2):
        sl = pl.ds(ob + r * 128, 128)
        d0 = pltpu.async_copy(tab0_hbm.at[sb.at[r]], gt0, sem)
        d1 = pltpu.async_copy(tab1_hbm.at[sb.at[r]], gt1, sem)
        d2 = pltpu.async_copy(skip_hbm.at[nb.at[r]], skt, sem)
        d0.wait()
        o0 = pltpu.async_copy(gt0, g0.at[sl, :], osem)
        d1.wait()
        o1 = pltpu.async_copy(gt1, g1.at[sl, :], osem)
        d2.wait()
        o2 = pltpu.async_copy(skt, sk.at[sl, :], osem)
        o0.wait()
        o1.wait()
        o2.wait()


_BLK7 = 512


def _k7_body(g0s, g1s, sks, g0d, g1d, skd, wsrc_ref, wdst_ref, bh_ref, wout_ref,
             bout_ref, y_ref):
    def node_out(a, b, sk):
        num = a[:, 0:128] + b[:, 0:128]
        d0 = a[:, 128:129] + b[:, 128:129]
        d1 = a[:, 129:130] + b[:, 129:130]
        den = jnp.concatenate(
            [jnp.broadcast_to(d0, (_BLK7, 64)), jnp.broadcast_to(d1, (_BLK7, 64))],
            axis=1)
        return num / (den + 1e-16) + sk

    os_ = node_out(g0s[...], g1s[...], sks[...])
    od_ = node_out(g0d[...], g1d[...], skd[...])
    h = os_ @ wsrc_ref[...] + od_ @ wdst_ref[...] + bh_ref[...]
    h = jnp.maximum(h, 0.0)
    y_ref[...] = h @ wout_ref[...] + bout_ref[...]


_k7 = pl.pallas_call(
    _k7_body,
    grid=(B // _BLK7,),
    in_specs=[
        pl.BlockSpec((_BLK7, PAYW), lambda i: (i, 0)),
        pl.BlockSpec((_BLK7, PAYW), lambda i: (i, 0)),
        pl.BlockSpec((_BLK7, 128), lambda i: (i, 0)),
        pl.BlockSpec((_BLK7, PAYW), lambda i: (i, 0)),
        pl.BlockSpec((_BLK7, PAYW), lambda i: (i, 0)),
        pl.BlockSpec((_BLK7, 128), lambda i: (i, 0)),
        pl.BlockSpec((128, 128), lambda i: (0, 0)),
        pl.BlockSpec((128, 128), lambda i: (0, 0)),
        pl.BlockSpec((1, 128), lambda i: (0, 0)),
        pl.BlockSpec((128, OUT_CH), lambda i: (0, 0)),
        pl.BlockSpec((1, OUT_CH), lambda i: (0, 0)),
    ],
    out_specs=pl.BlockSpec((_BLK7, OUT_CH), lambda i: (i, 0)),
    out_shape=jax.ShapeDtypeStruct((B, OUT_CH), jnp.float32),
)


def kernel(n_id, edge_index, t, msg, src, dst, memory, last_update, Wt, bt, Wq, bq, Wk, bk, Wv, bv, We, Wskip, bskip, Wsrc, Wdst, bh, Wout, bout):
    nid_p = jnp.concatenate([n_id, jnp.zeros((NPAD - N_BATCH,), jnp.int32)])
    srcdst = jnp.concatenate([src, dst])
    z, lu, slotmap = _k1(memory, last_update, nid_p, srcdst)

    w4 = jnp.concatenate([Wq, Wk, Wv, Wskip], axis=1)
    b4 = jnp.concatenate([bq, bk, bv, bskip])[None, :]
    q, kv, skip = _k2(z, w4, b4)

    epad = jnp.full((EPAD - N_EDGES,), N_BATCH, jnp.int32)
    esrc = jnp.concatenate([edge_index[0], epad])
    edst = jnp.concatenate([edge_index[1], epad])
    tp = jnp.concatenate([t, jnp.zeros((EPAD - N_EDGES,), jnp.int32)])
    ssrc, sdst, sslot, st_, seid, cnts = _k3a(esrc, edst, tp, slotmap)

    relt = _k3b1(ssrc, st_, lu)
    ssrc2 = ssrc.reshape(CAP // _GCH, _GCH)
    sdst2 = sdst.reshape(CAP // _GCH, _GCH)
    seid2 = seid.reshape(CAP // _GCH, _GCH)
    msgs, qs, kvs = _k3b2(ssrc2, sdst2, seid2, q, kv, msg, cnts)

    p = _k4(relt[:, None], msgs, kvs, qs, Wt, bt[None, :], We[:TIME_DIM], We[TIME_DIM:])

    tab0, tab1 = _k5(p, sslot, cnts)

    g0, g1, sk = _k6(slotmap, srcdst, tab0, tab1, skip)

    y = _k7(g0[:B], g1[:B], sk[:B], g0[B:], g1[B:], sk[B:],
            Wsrc, Wdst, bh[None, :], Wout, bout[None, :])
    return y
